# Initial kernel scaffold; baseline (speedup 1.0000x reference)
import functools

import jax
import jax.numpy as jnp
from jax import lax
from jax.experimental import pallas as pl
from jax.experimental.pallas import tpu as pltpu
from jax.experimental.pallas import tpu_sc as plsc

NC = 2
NS = 16
NW = NC * NS
LANES = 16
CH = 80

_MESH = dict(core_axis_name="c", subcore_axis_name="s")


def _proj_body(x_ref, wf_ref, bf_ref, o_ref):
    o_ref[...] = jnp.maximum(
        jnp.dot(x_ref[...], wf_ref[...], preferred_element_type=jnp.float32)
        + bf_ref[...], 0.0)


def _eprep_body(n_nodes, ea_ref, src_ref, dst_ref, g_ref, k_ref):
    ea = ea_ref[...]
    mx = ea[0:1, :]
    et = jnp.zeros(mx.shape, jnp.int32)
    for r in range(1, ea.shape[0]):
        row = ea[r:r + 1, :]
        gt = row > mx
        et = jnp.where(gt, r, et)
        mx = jnp.where(gt, row, mx)
    g_ref[...] = et * n_nodes + src_ref[...]
    k_ref[...] = et * n_nodes + dst_ref[...]


def _transform_body(h_ref, wr_ref, wroot_ref, bl_ref, h4_ref, base_ref):
    r = pl.program_id(1)
    hv = h_ref[...]

    @pl.when(r == 0)
    def _():
        base_ref[...] = jnp.dot(
            hv, wroot_ref[...], preferred_element_type=jnp.float32) + bl_ref[...]

    h4_ref[...] = jnp.dot(hv, wr_ref[0], preferred_element_type=jnp.float32)


def _combine_body(b_ref, p0_ref, p1_ref, o_ref):
    o_ref[...] = jnp.maximum(b_ref[...] + p0_ref[...] + p1_ref[...], 0.0)


def _make_hist_kernel(n4, eb):
    rows_per_tec = eb // NS
    zrows = 1250

    @functools.partial(
        pl.kernel,
        out_type=jax.ShapeDtypeStruct((n4, LANES), jnp.float32),
        mesh=plsc.VectorSubcoreMesh(**_MESH),
        scratch_types=[
            pltpu.VMEM((rows_per_tec, CH), jnp.int32),
            pltpu.VMEM((CH, LANES), jnp.float32),
            pltpu.VMEM((zrows, LANES), jnp.float32),
            pltpu.VMEM_SHARED((n4, LANES), jnp.float32),
        ],
    )
    def hist_kernel(key_hbm, cnt_hbm, kidx_v, ones_v, dz_v, acc):
        c = lax.axis_index("c")
        s = lax.axis_index("s")

        def initr(i, _):
            ones_v[i, :] = jnp.ones((LANES,), jnp.float32)
            return 0
        lax.fori_loop(0, CH, initr, 0)

        def zr(i, _):
            dz_v[i, :] = jnp.zeros((LANES,), jnp.float32)
            return 0
        lax.fori_loop(0, zrows, zr, 0)

        per_tec = n4 // NS
        for kk in range(per_tec // zrows):
            pltpu.sync_copy(dz_v, acc.at[pl.ds(s * per_tec + kk * zrows, zrows)])
        plsc.subcore_barrier()

        pltpu.sync_copy(key_hbm.at[pl.ds(s * rows_per_tec, rows_per_tec)], kidx_v)

        def chunk(i, _):
            pltpu.sync_copy(ones_v, acc.at[kidx_v.at[i]], add=True)
            return 0
        lax.fori_loop(0, rows_per_tec, chunk, 0)
        plsc.subcore_barrier()

        half = n4 // NC
        dper = half // NS
        for kk in range(dper // zrows):
            base = c * half + s * dper + kk * zrows
            pltpu.sync_copy(acc.at[pl.ds(base, zrows)], dz_v)
            pltpu.sync_copy(dz_v, cnt_hbm.at[pl.ds(base, zrows)])

    return hist_kernel


def _make_w_kernel(n4, eb, n_edges):
    rows_per_w = eb // NW

    @functools.partial(
        pl.kernel,
        out_type=jax.ShapeDtypeStruct((n_edges,), jnp.float32),
        mesh=plsc.VectorSubcoreMesh(**_MESH),
        scratch_types=[
            pltpu.VMEM((rows_per_w, CH), jnp.int32),
            pltpu.VMEM((CH, LANES), jnp.float32),
            pltpu.VMEM((CH,), jnp.float32),
            pltpu.SemaphoreType.DMA,
        ],
    )
    def w_kernel(key_hbm, cnt_hbm, w_hbm, kidx_v, crows_v, wbuf_v, sem):
        c = lax.axis_index("c")
        s = lax.axis_index("s")
        wid = s * NC + c
        pltpu.sync_copy(key_hbm.at[pl.ds(wid * rows_per_w, rows_per_w)], kidx_v)
        lane = lax.iota(jnp.int32, LANES)
        zeros16 = jnp.zeros((LANES,), jnp.int32)

        def chunk(i, _):
            pltpu.async_copy(cnt_hbm.at[kidx_v.at[i]], crows_v, sem).wait()

            def sub(j, _):
                ridx = j * LANES + lane
                c16 = plsc.load_gather(crows_v, [ridx, zeros16])
                wbuf_v[pl.ds(j * LANES, LANES)] = 1.0 / jnp.maximum(c16, 1.0)
                return 0
            lax.fori_loop(0, CH // LANES, sub, 0)
            pltpu.sync_copy(
                wbuf_v, w_hbm.at[pl.ds((wid * rows_per_w + i) * CH, CH)])
            return 0
        lax.fori_loop(0, rows_per_w, chunk, 0)

    return w_kernel


def _make_edge_kernel(n_nodes, h_dim, rows_per_w):
    e_per_w = rows_per_w * CH
    nsub = h_dim // LANES
    zrows = 125
    per_tec = n_nodes // NS

    @functools.partial(
        pl.kernel,
        out_type=(jax.ShapeDtypeStruct((n_nodes, h_dim), jnp.float32),
                  jax.ShapeDtypeStruct((n_nodes, h_dim), jnp.float32)),
        mesh=plsc.VectorSubcoreMesh(**_MESH),
        scratch_types=[
            pltpu.VMEM((rows_per_w, CH), jnp.int32),
            pltpu.VMEM((rows_per_w, CH), jnp.int32),
            pltpu.VMEM((e_per_w,), jnp.float32),
            pltpu.VMEM((CH, h_dim), jnp.float32),
            pltpu.VMEM((zrows, h_dim), jnp.float32),
            pltpu.SemaphoreType.DMA,
            pltpu.VMEM_SHARED((n_nodes, h_dim), jnp.float32),
        ],
    )
    def edge_kernel(h4_hbm, g_hbm, d_hbm, w_hbm, p0_hbm, p1_hbm,
                    gi_v, di_v, w_v, rows_v, drain_v, sem, acc):
        c = lax.axis_index("c")
        s = lax.axis_index("s")
        wid = s * NC + c

        pltpu.sync_copy(g_hbm.at[pl.ds(wid * rows_per_w, rows_per_w)], gi_v)
        pltpu.sync_copy(d_hbm.at[pl.ds(wid * rows_per_w, rows_per_w)], di_v)
        pltpu.sync_copy(w_hbm.at[pl.ds(wid * e_per_w, e_per_w)], w_v)

        def zr(i, _):
            for t in range(nsub):
                drain_v[i, pl.ds(t * LANES, LANES)] = jnp.zeros(
                    (LANES,), jnp.float32)
            return 0
        lax.fori_loop(0, zrows, zr, 0)
        for kk in range(per_tec // zrows):
            pltpu.sync_copy(
                drain_v, acc.at[pl.ds(s * per_tec + kk * zrows, zrows)])
        plsc.subcore_barrier()

        def chunk(i, _):
            pltpu.async_copy(h4_hbm.at[gi_v.at[i]], rows_v, sem).wait()

            def scale(j, _):
                widx = jnp.full((LANES,), i * CH + j, jnp.int32)
                wv = plsc.load_gather(w_v, [widx])
                for t in range(nsub):
                    rows_v[j, pl.ds(t * LANES, LANES)] = (
                        rows_v[j, pl.ds(t * LANES, LANES)] * wv)
                return 0
            lax.fori_loop(0, CH, scale, 0)
            pltpu.sync_copy(rows_v, acc.at[di_v.at[i]], add=True)
            return 0
        lax.fori_loop(0, rows_per_w, chunk, 0)
        plsc.subcore_barrier()

        for kk in range(per_tec // zrows):
            rbase = s * per_tec + kk * zrows
            pltpu.sync_copy(acc.at[pl.ds(rbase, zrows)], drain_v)

            @pl.when(c == 0)
            def _():
                pltpu.sync_copy(drain_v, p0_hbm.at[pl.ds(rbase, zrows)])

            @pl.when(c == 1)
            def _():
                pltpu.sync_copy(drain_v, p1_hbm.at[pl.ds(rbase, zrows)])

    return edge_kernel


def kernel(x, edge_index, edge_attr, W_f, b_f, W_rel, W_root, b):
    n, d = x.shape
    h_dim = W_f.shape[1]
    n_edges = edge_index.shape[1]
    n_rel = edge_attr.shape[1]
    n_layers = W_rel.shape[0]
    n4 = n_rel * n
    eb = n_edges // CH
    rows_per_w = eb // NW
    bn = 1000
    nb = n // bn
    be = 6400
    f32 = jnp.float32

    src = edge_index[0].reshape(1, n_edges)
    dst = edge_index[1].reshape(1, n_edges)
    ea_t = edge_attr.T

    proj = pl.pallas_call(
        _proj_body,
        grid=(nb,),
        in_specs=[
            pl.BlockSpec((bn, d), lambda i: (i, 0)),
            pl.BlockSpec((d, h_dim), lambda i: (0, 0)),
            pl.BlockSpec((1, h_dim), lambda i: (0, 0)),
        ],
        out_specs=pl.BlockSpec((bn, h_dim), lambda i: (i, 0)),
        out_shape=jax.ShapeDtypeStruct((n, h_dim), f32),
    )
    h = proj(x, W_f, b_f.reshape(1, h_dim))

    eprep = pl.pallas_call(
        functools.partial(_eprep_body, n),
        grid=(n_edges // be,),
        in_specs=[
            pl.BlockSpec((n_rel, be), lambda i: (0, i)),
            pl.BlockSpec((1, be), lambda i: (0, i)),
            pl.BlockSpec((1, be), lambda i: (0, i)),
        ],
        out_specs=[
            pl.BlockSpec((1, be), lambda i: (0, i)),
            pl.BlockSpec((1, be), lambda i: (0, i)),
        ],
        out_shape=[jax.ShapeDtypeStruct((1, n_edges), jnp.int32),
                   jax.ShapeDtypeStruct((1, n_edges), jnp.int32)],
    )
    g2, k2 = eprep(ea_t, src, dst)
    g2 = g2.reshape(eb, CH)
    k2 = k2.reshape(eb, CH)
    d2 = dst.reshape(eb, CH)

    cnt = _make_hist_kernel(n4, eb)(k2)
    w = _make_w_kernel(n4, eb, n_edges)(k2, cnt)

    transform = pl.pallas_call(
        _transform_body,
        grid=(nb, n_rel),
        in_specs=[
            pl.BlockSpec((bn, h_dim), lambda i, r: (i, 0)),
            pl.BlockSpec((1, h_dim, h_dim), lambda i, r: (r, 0, 0)),
            pl.BlockSpec((h_dim, h_dim), lambda i, r: (0, 0)),
            pl.BlockSpec((1, h_dim), lambda i, r: (0, 0)),
        ],
        out_specs=[
            pl.BlockSpec((bn, h_dim), lambda i, r: (r * nb + i, 0)),
            pl.BlockSpec((bn, h_dim), lambda i, r: (i, 0)),
        ],
        out_shape=[jax.ShapeDtypeStruct((n4, h_dim), f32),
                   jax.ShapeDtypeStruct((n, h_dim), f32)],
    )

    combine = pl.pallas_call(
        _combine_body,
        grid=(nb,),
        in_specs=[pl.BlockSpec((bn, h_dim), lambda i: (i, 0))] * 3,
        out_specs=pl.BlockSpec((bn, h_dim), lambda i: (i, 0)),
        out_shape=jax.ShapeDtypeStruct((n, h_dim), f32),
    )

    edge_pass = _make_edge_kernel(n, h_dim, rows_per_w)

    for l in range(n_layers):
        h4, base = transform(h, W_rel[l], W_root[l], b[l].reshape(1, h_dim))
        p0, p1 = edge_pass(h4, g2, d2, w)
        h = combine(base, p0, p1)
    return h


# speedup vs baseline: 2.3072x; 2.3072x over previous
import functools

import jax
import jax.numpy as jnp
from jax import lax
from jax.experimental import pallas as pl
from jax.experimental.pallas import tpu as pltpu
from jax.experimental.pallas import tpu_sc as plsc

NC = 2
NS = 16
NW = NC * NS
LANES = 16
CH = 80
BLK = 800
NBLK_CAP = 14
REG_CAP = NBLK_CAP * BLK

_MESH = dict(core_axis_name="c", subcore_axis_name="s")
_SC_PARAMS = pltpu.CompilerParams(
    use_tc_tiling_on_sc=False, needs_layout_passes=False)


def _proj_body(x_ref, wf_ref, bf_ref, o_ref):
    o_ref[...] = jnp.maximum(
        jnp.dot(x_ref[...], wf_ref[...], preferred_element_type=jnp.float32)
        + bf_ref[...], 0.0)


def _eprep_body(n_nodes, ea_ref, src_ref, dst_ref, g_ref, k_ref):
    ea = ea_ref[...]
    mx = ea[0:1, :]
    et = jnp.zeros(mx.shape, jnp.int32)
    for r in range(1, ea.shape[0]):
        row = ea[r:r + 1, :]
        gt = row > mx
        et = jnp.where(gt, r, et)
        mx = jnp.where(gt, row, mx)
    g_ref[...] = et * n_nodes + src_ref[...]
    k_ref[...] = et * n_nodes + dst_ref[...]


def _transform_body(h_ref, wr_ref, wroot_ref, bl_ref, h4_ref, base_ref):
    r = pl.program_id(1)
    hv = h_ref[...]

    @pl.when(r == 0)
    def _():
        base_ref[...] = jnp.dot(
            hv, wroot_ref[...], preferred_element_type=jnp.float32) + bl_ref[...]

    h4_ref[...] = jnp.dot(hv, wr_ref[0], preferred_element_type=jnp.float32)


def _combine_body(b_ref, p_ref, o_ref):
    o_ref[...] = jnp.maximum(b_ref[...] + p_ref[...], 0.0)


def _make_hist_kernel(n4, eb):
    rows_per_tec = eb // NS
    zrows = n4 // NW

    @functools.partial(
        pl.kernel,
        out_type=jax.ShapeDtypeStruct((NW, zrows, LANES), jnp.float32),
        mesh=plsc.VectorSubcoreMesh(**_MESH),
        compiler_params=_SC_PARAMS,
        scratch_types=[
            pltpu.VMEM((rows_per_tec, CH), jnp.int32),
            pltpu.VMEM((CH, LANES), jnp.float32),
            pltpu.VMEM((zrows, LANES), jnp.float32),
            pltpu.VMEM_SHARED((n4, LANES), jnp.float32),
        ],
    )
    def hist_kernel(key_hbm, cnt_hbm, kidx_v, ones_v, dz_v, acc):
        c = lax.axis_index("c")
        s = lax.axis_index("s")

        def initr(i, _):
            ones_v[i, :] = jnp.ones((LANES,), jnp.float32)
            return 0
        lax.fori_loop(0, CH, initr, 0)

        def zr(i, _):
            dz_v[i, :] = jnp.zeros((LANES,), jnp.float32)
            return 0
        lax.fori_loop(0, zrows, zr, 0)

        per_tec = n4 // NS
        for kk in range(per_tec // zrows):
            pltpu.sync_copy(dz_v, acc.at[pl.ds(s * per_tec + kk * zrows, zrows)])
        plsc.subcore_barrier()

        pltpu.sync_copy(key_hbm.at[s], kidx_v)

        def chunk(i, _):
            pltpu.sync_copy(ones_v, acc.at[kidx_v.at[i]], add=True)
            return 0
        lax.fori_loop(0, rows_per_tec, chunk, 0)
        plsc.subcore_barrier()

        dd = c * NS + s
        pltpu.sync_copy(acc.at[pl.ds(dd * zrows, zrows)], dz_v)
        pltpu.sync_copy(dz_v, cnt_hbm.at[dd])

    return hist_kernel


def _make_w_kernel(n4, eb, n_edges):
    rows_per_w = eb // NW

    @functools.partial(
        pl.kernel,
        out_type=jax.ShapeDtypeStruct((n_edges,), jnp.float32),
        mesh=plsc.VectorSubcoreMesh(**_MESH),
        compiler_params=_SC_PARAMS,
        scratch_types=[
            pltpu.VMEM((rows_per_w, CH), jnp.int32),
            pltpu.VMEM((CH, LANES), jnp.float32),
            pltpu.VMEM((CH,), jnp.float32),
            pltpu.SemaphoreType.DMA,
        ],
    )
    def w_kernel(key_hbm, cnt_hbm, w_hbm, kidx_v, crows_v, wbuf_v, sem):
        c = lax.axis_index("c")
        s = lax.axis_index("s")
        wid = s * NC + c
        pltpu.sync_copy(key_hbm.at[wid], kidx_v)
        lane = lax.iota(jnp.int32, LANES)
        zeros16 = jnp.zeros((LANES,), jnp.int32)

        def chunk(i, _):
            pltpu.async_copy(cnt_hbm.at[kidx_v.at[i]], crows_v, sem).wait()

            def sub(j, _):
                ridx = j * LANES + lane
                c16 = plsc.load_gather(crows_v, [ridx, zeros16])
                wbuf_v[pl.ds(j * LANES, LANES)] = 1.0 / jnp.maximum(c16, 1.0)
                return 0
            lax.fori_loop(0, CH // LANES, sub, 0)
            pltpu.sync_copy(
                wbuf_v, w_hbm.at[pl.ds((wid * rows_per_w + i) * CH, CH)])
            return 0
        lax.fori_loop(0, rows_per_w, chunk, 0)

    return w_kernel


def _make_partition_kernel(n_nodes, n_edges):
    e_per_w = n_edges // NW
    half = n_nodes // NC
    groups = e_per_w // LANES

    @functools.partial(
        pl.kernel,
        out_type=(jax.ShapeDtypeStruct((NC, NW, REG_CAP), jnp.int32),
                  jax.ShapeDtypeStruct((NC, NW, REG_CAP), jnp.int32),
                  jax.ShapeDtypeStruct((NC, NW, REG_CAP), jnp.float32),
                  jax.ShapeDtypeStruct((NC, NW, LANES), jnp.int32)),
        mesh=plsc.VectorSubcoreMesh(**_MESH),
        compiler_params=_SC_PARAMS,
        scratch_types=[
            pltpu.VMEM((e_per_w,), jnp.int32),
            pltpu.VMEM((e_per_w,), jnp.int32),
            pltpu.VMEM((e_per_w,), jnp.float32),
            pltpu.VMEM((NC, REG_CAP), jnp.int32),
            pltpu.VMEM((NC, REG_CAP), jnp.int32),
            pltpu.VMEM((NC, REG_CAP), jnp.float32),
            pltpu.VMEM((LANES,), jnp.int32),
        ],
    )
    def part_kernel(g_hbm, d_hbm, w_hbm,
                    og_hbm, od_hbm, ow_hbm, nb_hbm,
                    gb_v, db_v, wb_v, og_v, od_v, ow_v, nb_v):
        c = lax.axis_index("c")
        s = lax.axis_index("s")
        wid = s * NC + c

        pltpu.sync_copy(g_hbm.at[wid], gb_v)
        pltpu.sync_copy(d_hbm.at[wid], db_v)
        pltpu.sync_copy(w_hbm.at[wid], wb_v)

        def group(k, offs):
            o0, o1 = offs
            g16 = gb_v[pl.ds(k * LANES, LANES)]
            d16 = db_v[pl.ds(k * LANES, LANES)]
            w16 = wb_v[pl.ds(k * LANES, LANES)]
            m0 = d16 < half
            m1 = jnp.logical_not(m0)
            plsc.store_compressed(og_v.at[0, pl.ds(o0, LANES)], g16, mask=m0)
            plsc.store_compressed(od_v.at[0, pl.ds(o0, LANES)], d16, mask=m0)
            plsc.store_compressed(ow_v.at[0, pl.ds(o0, LANES)], w16, mask=m0)
            n0 = lax.reduce_max(
                plsc.all_reduce_population_count(m0), axes=(0,))
            plsc.store_compressed(og_v.at[1, pl.ds(o1, LANES)], g16, mask=m1)
            plsc.store_compressed(od_v.at[1, pl.ds(o1, LANES)], d16 - half, mask=m1)
            plsc.store_compressed(ow_v.at[1, pl.ds(o1, LANES)], w16, mask=m1)
            n1 = lax.reduce_max(
                plsc.all_reduce_population_count(m1), axes=(0,))
            return (o0 + n0, o1 + n1)

        o0, o1 = lax.fori_loop(0, groups, group, (jnp.int32(0), jnp.int32(0)))

        zi16 = jnp.zeros((LANES,), jnp.int32)
        dummyd = jnp.full((LANES,), half, jnp.int32)
        zf16 = jnp.zeros((LANES,), jnp.float32)
        for hh, off in ((0, o0), (1, o1)):
            for i in range(BLK // LANES):
                og_v[hh, pl.ds(off + i * LANES, LANES)] = zi16
                od_v[hh, pl.ds(off + i * LANES, LANES)] = dummyd
                ow_v[hh, pl.ds(off + i * LANES, LANES)] = zf16

        pltpu.sync_copy(og_v.at[0], og_hbm.at[0, wid])
        pltpu.sync_copy(od_v.at[0], od_hbm.at[0, wid])
        pltpu.sync_copy(ow_v.at[0], ow_hbm.at[0, wid])
        pltpu.sync_copy(og_v.at[1], og_hbm.at[1, wid])
        pltpu.sync_copy(od_v.at[1], od_hbm.at[1, wid])
        pltpu.sync_copy(ow_v.at[1], ow_hbm.at[1, wid])

        for hh, off in ((0, o0), (1, o1)):
            nblk = (off + (BLK - 1)) // BLK
            nb_v[:] = jnp.broadcast_to(nblk, (LANES,)).astype(jnp.int32)
            pltpu.sync_copy(nb_v, nb_hbm.at[hh, wid])

    return part_kernel


def _make_edge_kernel(n_nodes, h_dim):
    half = n_nodes // NC
    acc_rows = ((half // (NS * CH)) + 1) * NS * CH
    nsub = h_dim // LANES
    drows = acc_rows // NS // CH

    @functools.partial(
        pl.kernel,
        out_type=(jax.ShapeDtypeStruct((NS * drows, CH, h_dim), jnp.float32),
                  jax.ShapeDtypeStruct((NS * drows, CH, h_dim), jnp.float32)),
        mesh=plsc.VectorSubcoreMesh(**_MESH),
        compiler_params=_SC_PARAMS,
        scratch_types=[
            pltpu.VMEM((BLK,), jnp.int32),
            pltpu.VMEM((BLK,), jnp.int32),
            pltpu.VMEM((BLK // CH, CH), jnp.int32),
            pltpu.VMEM((BLK,), jnp.float32),
            pltpu.VMEM((LANES,), jnp.int32),
            pltpu.VMEM((CH, h_dim), jnp.float32),
            pltpu.VMEM((CH, h_dim), jnp.float32),
            pltpu.SemaphoreType.DMA,
            pltpu.VMEM_SHARED((acc_rows, h_dim), jnp.float32),
        ],
    )
    def edge_kernel(h4_hbm, og_hbm, od_hbm, ow_hbm, nb_hbm, p0_hbm, p1_hbm,
                    gf_v, df_v, di_v, w_v, nb_v, rows_v, drain_v, sem, acc):
        c = lax.axis_index("c")
        s = lax.axis_index("s")

        def zr(i, _):
            for t in range(nsub):
                drain_v[i, pl.ds(t * LANES, LANES)] = jnp.zeros(
                    (LANES,), jnp.float32)
            return 0
        lax.fori_loop(0, CH, zr, 0)
        for kk in range(drows):
            pltpu.sync_copy(
                drain_v, acc.at[pl.ds((s * drows + kk) * CH, CH)])
        plsc.subcore_barrier()

        for ridx in range(NW // NS):
            rg = s + NS * ridx
            pltpu.sync_copy(nb_hbm.at[c, rg], nb_v)
            nblk = lax.reduce_max(nb_v[:], axes=(0,))

            def block(bi, _):
                base = bi * BLK
                pltpu.sync_copy(og_hbm.at[c, rg, pl.ds(base, BLK)], gf_v)
                pltpu.sync_copy(od_hbm.at[c, rg, pl.ds(base, BLK)], df_v)
                pltpu.sync_copy(ow_hbm.at[c, rg, pl.ds(base, BLK)], w_v)
                for i in range(BLK // CH):
                    for j in range(CH // LANES):
                        di_v[i, pl.ds(j * LANES, LANES)] = (
                            df_v[pl.ds(i * CH + j * LANES, LANES)])

                def chunk(k, _):
                    pltpu.async_copy(
                        h4_hbm.at[gf_v.at[pl.ds(k * CH, CH)]], rows_v,
                        sem).wait()

                    def scale(j2, _):
                        wv = plsc.load_gather(
                            w_v, [jnp.full((LANES,), k * CH + j2, jnp.int32)])
                        for t in range(nsub):
                            rows_v[j2, pl.ds(t * LANES, LANES)] = (
                                rows_v[j2, pl.ds(t * LANES, LANES)] * wv)
                        return 0
                    lax.fori_loop(0, CH, scale, 0)
                    pltpu.sync_copy(rows_v, acc.at[di_v.at[k]], add=True)
                    return 0
                lax.fori_loop(0, BLK // CH, chunk, 0)
                return 0
            lax.fori_loop(0, nblk, block, 0)

        plsc.subcore_barrier()
        for kk in range(drows):
            pltpu.sync_copy(acc.at[pl.ds((s * drows + kk) * CH, CH)], drain_v)

            @pl.when(c == 0)
            def _():
                pltpu.sync_copy(drain_v, p0_hbm.at[s * drows + kk])

            @pl.when(c == 1)
            def _():
                pltpu.sync_copy(drain_v, p1_hbm.at[s * drows + kk])

    return edge_kernel


def kernel(x, edge_index, edge_attr, W_f, b_f, W_rel, W_root, b):
    n, d = x.shape
    h_dim = W_f.shape[1]
    n_edges = edge_index.shape[1]
    n_rel = edge_attr.shape[1]
    n_layers = W_rel.shape[0]
    n4 = n_rel * n
    eb = n_edges // CH
    half = n // NC
    bn = 1000
    nb = n // bn
    be = 6400
    f32 = jnp.float32

    src = edge_index[0].reshape(1, n_edges)
    dst = edge_index[1].reshape(1, n_edges)
    ea_t = edge_attr.T

    proj = pl.pallas_call(
        _proj_body,
        grid=(nb,),
        in_specs=[
            pl.BlockSpec((bn, d), lambda i: (i, 0)),
            pl.BlockSpec((d, h_dim), lambda i: (0, 0)),
            pl.BlockSpec((1, h_dim), lambda i: (0, 0)),
        ],
        out_specs=pl.BlockSpec((bn, h_dim), lambda i: (i, 0)),
        out_shape=jax.ShapeDtypeStruct((n, h_dim), f32),
    )
    h = proj(x, W_f, b_f.reshape(1, h_dim))

    eprep = pl.pallas_call(
        functools.partial(_eprep_body, n),
        grid=(n_edges // be,),
        in_specs=[
            pl.BlockSpec((n_rel, be), lambda i: (0, i)),
            pl.BlockSpec((1, be), lambda i: (0, i)),
            pl.BlockSpec((1, be), lambda i: (0, i)),
        ],
        out_specs=[
            pl.BlockSpec((1, be), lambda i: (0, i)),
            pl.BlockSpec((1, be), lambda i: (0, i)),
        ],
        out_shape=[jax.ShapeDtypeStruct((1, n_edges), jnp.int32),
                   jax.ShapeDtypeStruct((1, n_edges), jnp.int32)],
    )
    g2, k2 = eprep(ea_t, src, dst)
    k16 = k2.reshape(NS, eb // NS, CH)
    k32 = k2.reshape(NW, eb // NW, CH)
    g32 = g2.reshape(NW, n_edges // NW)
    d32 = dst.reshape(NW, n_edges // NW)

    cnt = _make_hist_kernel(n4, eb)(k16).reshape(n4, LANES)
    w = _make_w_kernel(n4, eb, n_edges)(k32, cnt)
    w32 = w.reshape(NW, n_edges // NW)

    og, od, ow, nblk = _make_partition_kernel(n, n_edges)(g32, d32, w32)

    transform = pl.pallas_call(
        _transform_body,
        grid=(nb, n_rel),
        in_specs=[
            pl.BlockSpec((bn, h_dim), lambda i, r: (i, 0)),
            pl.BlockSpec((1, h_dim, h_dim), lambda i, r: (r, 0, 0)),
            pl.BlockSpec((h_dim, h_dim), lambda i, r: (0, 0)),
            pl.BlockSpec((1, h_dim), lambda i, r: (0, 0)),
        ],
        out_specs=[
            pl.BlockSpec((bn, h_dim), lambda i, r: (r * nb + i, 0)),
            pl.BlockSpec((bn, h_dim), lambda i, r: (i, 0)),
        ],
        out_shape=[jax.ShapeDtypeStruct((n4, h_dim), f32),
                   jax.ShapeDtypeStruct((n, h_dim), f32)],
    )

    combine = pl.pallas_call(
        _combine_body,
        grid=(nb,),
        in_specs=[pl.BlockSpec((bn, h_dim), lambda i: (i, 0))] * 2,
        out_specs=pl.BlockSpec((bn, h_dim), lambda i: (i, 0)),
        out_shape=jax.ShapeDtypeStruct((n, h_dim), f32),
    )

    edge_pass = _make_edge_kernel(n, h_dim)

    for l in range(n_layers):
        h4, base = transform(h, W_rel[l], W_root[l], b[l].reshape(1, h_dim))
        p0, p1 = edge_pass(h4, og, od, ow, nblk)
        p = jnp.concatenate(
            [p0.reshape(-1, h_dim)[:half], p1.reshape(-1, h_dim)[:half]], 0)
        h = combine(base, p)
    return h


# speedup vs baseline: 2.4424x; 1.0586x over previous
import functools

import jax
import jax.numpy as jnp
from jax import lax
from jax.experimental import pallas as pl
from jax.experimental.pallas import tpu as pltpu
from jax.experimental.pallas import tpu_sc as plsc

NC = 2
NS = 16
NW = NC * NS
LANES = 16
CH = 80
BLK = 800
NBLK_CAP = 14
REG_CAP = NBLK_CAP * BLK

_MESH = dict(core_axis_name="c", subcore_axis_name="s")
_SC_PARAMS = pltpu.CompilerParams(
    use_tc_tiling_on_sc=False, needs_layout_passes=False)


def _proj_body(x_ref, wf_ref, bf_ref, o_ref):
    o_ref[...] = jnp.maximum(
        jnp.dot(x_ref[...], wf_ref[...], preferred_element_type=jnp.float32)
        + bf_ref[...], 0.0)


def _eprep_body(n_nodes, ea_ref, src_ref, dst_ref, g_ref, k_ref):
    ea = ea_ref[...]
    mx = ea[0:1, :]
    et = jnp.zeros(mx.shape, jnp.int32)
    for r in range(1, ea.shape[0]):
        row = ea[r:r + 1, :]
        gt = row > mx
        et = jnp.where(gt, r, et)
        mx = jnp.where(gt, row, mx)
    g_ref[...] = et * n_nodes + src_ref[...]
    k_ref[...] = et * n_nodes + dst_ref[...]


def _transform_body(h_ref, wr_ref, wroot_ref, bl_ref, h4_ref, base_ref):
    r = pl.program_id(1)
    hv = h_ref[...]

    @pl.when(r == 0)
    def _():
        base_ref[...] = jnp.dot(
            hv, wroot_ref[...], preferred_element_type=jnp.float32) + bl_ref[...]

    h4_ref[...] = jnp.dot(hv, wr_ref[0], preferred_element_type=jnp.float32)


def _combine_body(b_ref, p_ref, o_ref):
    o_ref[...] = jnp.maximum(b_ref[...] + p_ref[...], 0.0)


def _make_hist_kernel(n4, eb):
    rows_per_tec = eb // NS
    zrows = n4 // NW

    @functools.partial(
        pl.kernel,
        out_type=jax.ShapeDtypeStruct((NW, zrows, LANES), jnp.float32),
        mesh=plsc.VectorSubcoreMesh(**_MESH),
        compiler_params=_SC_PARAMS,
        scratch_types=[
            pltpu.VMEM((rows_per_tec, CH), jnp.int32),
            pltpu.VMEM((CH, LANES), jnp.float32),
            pltpu.VMEM((zrows, LANES), jnp.float32),
            pltpu.VMEM_SHARED((n4, LANES), jnp.float32),
        ],
    )
    def hist_kernel(key_hbm, cnt_hbm, kidx_v, ones_v, dz_v, acc):
        c = lax.axis_index("c")
        s = lax.axis_index("s")

        def initr(i, _):
            ones_v[i, :] = jnp.ones((LANES,), jnp.float32)
            return 0
        lax.fori_loop(0, CH, initr, 0)

        def zr(i, _):
            dz_v[i, :] = jnp.zeros((LANES,), jnp.float32)
            return 0
        lax.fori_loop(0, zrows, zr, 0)

        per_tec = n4 // NS
        for kk in range(per_tec // zrows):
            pltpu.sync_copy(dz_v, acc.at[pl.ds(s * per_tec + kk * zrows, zrows)])
        plsc.subcore_barrier()

        pltpu.sync_copy(key_hbm.at[s], kidx_v)

        def chunk(i, _):
            pltpu.sync_copy(ones_v, acc.at[kidx_v.at[i]], add=True)
            return 0
        lax.fori_loop(0, rows_per_tec, chunk, 0)
        plsc.subcore_barrier()

        dd = c * NS + s
        pltpu.sync_copy(acc.at[pl.ds(dd * zrows, zrows)], dz_v)
        pltpu.sync_copy(dz_v, cnt_hbm.at[dd])

    return hist_kernel


def _make_w_kernel(n4, eb, n_edges):
    rows_per_w = eb // NW

    @functools.partial(
        pl.kernel,
        out_type=jax.ShapeDtypeStruct((n_edges,), jnp.float32),
        mesh=plsc.VectorSubcoreMesh(**_MESH),
        compiler_params=_SC_PARAMS,
        scratch_types=[
            pltpu.VMEM((rows_per_w, CH), jnp.int32),
            pltpu.VMEM((CH, LANES), jnp.float32),
            pltpu.VMEM((CH,), jnp.float32),
            pltpu.SemaphoreType.DMA,
        ],
    )
    def w_kernel(key_hbm, cnt_hbm, w_hbm, kidx_v, crows_v, wbuf_v, sem):
        c = lax.axis_index("c")
        s = lax.axis_index("s")
        wid = s * NC + c
        pltpu.sync_copy(key_hbm.at[wid], kidx_v)
        lane = lax.iota(jnp.int32, LANES)
        zeros16 = jnp.zeros((LANES,), jnp.int32)

        def chunk(i, _):
            pltpu.async_copy(cnt_hbm.at[kidx_v.at[i]], crows_v, sem).wait()

            def sub(j, _):
                ridx = j * LANES + lane
                c16 = plsc.load_gather(crows_v, [ridx, zeros16])
                wbuf_v[pl.ds(j * LANES, LANES)] = 1.0 / jnp.maximum(c16, 1.0)
                return 0
            lax.fori_loop(0, CH // LANES, sub, 0)
            pltpu.sync_copy(
                wbuf_v, w_hbm.at[pl.ds((wid * rows_per_w + i) * CH, CH)])
            return 0
        lax.fori_loop(0, rows_per_w, chunk, 0)

    return w_kernel


def _make_partition_kernel(n_nodes, n_edges):
    e_per_w = n_edges // NW
    half = n_nodes // NC
    groups = e_per_w // LANES

    @functools.partial(
        pl.kernel,
        out_type=(jax.ShapeDtypeStruct((NC, NW, REG_CAP), jnp.int32),
                  jax.ShapeDtypeStruct((NC, NW, REG_CAP), jnp.int32),
                  jax.ShapeDtypeStruct((NC, NW, REG_CAP), jnp.float32),
                  jax.ShapeDtypeStruct((NC, NW, LANES), jnp.int32)),
        mesh=plsc.VectorSubcoreMesh(**_MESH),
        compiler_params=_SC_PARAMS,
        scratch_types=[
            pltpu.VMEM((e_per_w,), jnp.int32),
            pltpu.VMEM((e_per_w,), jnp.int32),
            pltpu.VMEM((e_per_w,), jnp.float32),
            pltpu.VMEM((NC, REG_CAP), jnp.int32),
            pltpu.VMEM((NC, REG_CAP), jnp.int32),
            pltpu.VMEM((NC, REG_CAP), jnp.float32),
            pltpu.VMEM((LANES,), jnp.int32),
        ],
    )
    def part_kernel(g_hbm, d_hbm, w_hbm,
                    og_hbm, od_hbm, ow_hbm, nb_hbm,
                    gb_v, db_v, wb_v, og_v, od_v, ow_v, nb_v):
        c = lax.axis_index("c")
        s = lax.axis_index("s")
        wid = s * NC + c

        pltpu.sync_copy(g_hbm.at[wid], gb_v)
        pltpu.sync_copy(d_hbm.at[wid], db_v)
        pltpu.sync_copy(w_hbm.at[wid], wb_v)

        def group(k, offs):
            o0, o1 = offs
            g16 = gb_v[pl.ds(k * LANES, LANES)]
            d16 = db_v[pl.ds(k * LANES, LANES)]
            w16 = wb_v[pl.ds(k * LANES, LANES)]
            m0 = d16 < half
            m1 = jnp.logical_not(m0)
            plsc.store_compressed(og_v.at[0, pl.ds(o0, LANES)], g16, mask=m0)
            plsc.store_compressed(od_v.at[0, pl.ds(o0, LANES)], d16, mask=m0)
            plsc.store_compressed(ow_v.at[0, pl.ds(o0, LANES)], w16, mask=m0)
            n0 = lax.reduce_max(
                plsc.all_reduce_population_count(m0), axes=(0,))
            plsc.store_compressed(og_v.at[1, pl.ds(o1, LANES)], g16, mask=m1)
            plsc.store_compressed(od_v.at[1, pl.ds(o1, LANES)], d16 - half, mask=m1)
            plsc.store_compressed(ow_v.at[1, pl.ds(o1, LANES)], w16, mask=m1)
            n1 = lax.reduce_max(
                plsc.all_reduce_population_count(m1), axes=(0,))
            return (o0 + n0, o1 + n1)

        o0, o1 = lax.fori_loop(0, groups, group, (jnp.int32(0), jnp.int32(0)))

        zi16 = jnp.zeros((LANES,), jnp.int32)
        dummyd = jnp.full((LANES,), half, jnp.int32)
        zf16 = jnp.zeros((LANES,), jnp.float32)
        for hh, off in ((0, o0), (1, o1)):
            for i in range(BLK // LANES):
                og_v[hh, pl.ds(off + i * LANES, LANES)] = zi16
                od_v[hh, pl.ds(off + i * LANES, LANES)] = dummyd
                ow_v[hh, pl.ds(off + i * LANES, LANES)] = zf16

        pltpu.sync_copy(og_v.at[0], og_hbm.at[0, wid])
        pltpu.sync_copy(od_v.at[0], od_hbm.at[0, wid])
        pltpu.sync_copy(ow_v.at[0], ow_hbm.at[0, wid])
        pltpu.sync_copy(og_v.at[1], og_hbm.at[1, wid])
        pltpu.sync_copy(od_v.at[1], od_hbm.at[1, wid])
        pltpu.sync_copy(ow_v.at[1], ow_hbm.at[1, wid])

        for hh, off in ((0, o0), (1, o1)):
            nblk = (off + (BLK - 1)) // BLK
            nb_v[:] = jnp.broadcast_to(nblk, (LANES,)).astype(jnp.int32)
            pltpu.sync_copy(nb_v, nb_hbm.at[hh, wid])

    return part_kernel


def _make_edge_kernel(n_nodes, h_dim):
    half = n_nodes // NC
    acc_rows = ((half // (NS * CH)) + 1) * NS * CH
    nsub = h_dim // LANES
    drows = acc_rows // NS // CH

    @functools.partial(
        pl.kernel,
        out_type=(jax.ShapeDtypeStruct((NS * drows, CH, h_dim), jnp.float32),
                  jax.ShapeDtypeStruct((NS * drows, CH, h_dim), jnp.float32)),
        mesh=plsc.VectorSubcoreMesh(**_MESH),
        compiler_params=_SC_PARAMS,
        scratch_types=[
            pltpu.VMEM((BLK,), jnp.int32),
            pltpu.VMEM((BLK,), jnp.int32),
            pltpu.VMEM((BLK // CH, CH), jnp.int32),
            pltpu.VMEM((BLK,), jnp.float32),
            pltpu.VMEM((LANES,), jnp.int32),
            pltpu.VMEM((CH, h_dim), jnp.float32),
            pltpu.VMEM((CH, h_dim), jnp.float32),
            pltpu.VMEM((CH, h_dim), jnp.float32),
            pltpu.SemaphoreType.DMA,
            pltpu.SemaphoreType.DMA,
            pltpu.VMEM_SHARED((acc_rows, h_dim), jnp.float32),
        ],
    )
    def edge_kernel(h4_hbm, og_hbm, od_hbm, ow_hbm, nb_hbm, p0_hbm, p1_hbm,
                    gf_v, df_v, di_v, w_v, nb_v, rows0_v, rows1_v, drain_v,
                    semg, sems, acc):
        c = lax.axis_index("c")
        s = lax.axis_index("s")

        def zr(i, _):
            for t in range(nsub):
                drain_v[i, pl.ds(t * LANES, LANES)] = jnp.zeros(
                    (LANES,), jnp.float32)
            return 0
        lax.fori_loop(0, CH, zr, 0)
        for kk in range(drows):
            pltpu.sync_copy(
                drain_v, acc.at[pl.ds((s * drows + kk) * CH, CH)])
        plsc.subcore_barrier()

        for ridx in range(NW // NS):
            rg = s + NS * ridx
            pltpu.sync_copy(nb_hbm.at[c, rg], nb_v)
            nblk = lax.reduce_max(nb_v[:], axes=(0,))

            nch = BLK // CH
            rows = (rows0_v, rows1_v)

            def block(bi, _):
                base = bi * BLK
                pltpu.sync_copy(og_hbm.at[c, rg, pl.ds(base, BLK)], gf_v)
                pltpu.sync_copy(od_hbm.at[c, rg, pl.ds(base, BLK)], df_v)
                pltpu.sync_copy(ow_hbm.at[c, rg, pl.ds(base, BLK)], w_v)
                for i in range(nch):
                    for j in range(CH // LANES):
                        di_v[i, pl.ds(j * LANES, LANES)] = (
                            df_v[pl.ds(i * CH + j * LANES, LANES)])

                descs_g = [None] * nch
                descs_s = [None] * nch
                descs_g[0] = pltpu.async_copy(
                    h4_hbm.at[gf_v.at[pl.ds(0, CH)]], rows[0], semg)
                for k in range(nch):
                    descs_g[k].wait()
                    if k + 1 < nch:
                        if k - 1 >= 0:
                            descs_s[k - 1].wait()
                        descs_g[k + 1] = pltpu.async_copy(
                            h4_hbm.at[gf_v.at[pl.ds((k + 1) * CH, CH)]],
                            rows[(k + 1) % 2], semg)
                    rbuf = rows[k % 2]

                    def scale(j2, _, k=k, rbuf=rbuf):
                        wv = plsc.load_gather(
                            w_v, [jnp.full((LANES,), k * CH + j2, jnp.int32)])
                        for t in range(nsub):
                            rbuf[j2, pl.ds(t * LANES, LANES)] = (
                                rbuf[j2, pl.ds(t * LANES, LANES)] * wv)
                        return 0
                    lax.fori_loop(0, CH, scale, 0)
                    descs_s[k] = pltpu.async_copy(
                        rbuf, acc.at[di_v.at[k]], sems, add=True)
                descs_s[nch - 2].wait()
                descs_s[nch - 1].wait()
                return 0
            lax.fori_loop(0, nblk, block, 0)

        plsc.subcore_barrier()
        for kk in range(drows):
            pltpu.sync_copy(acc.at[pl.ds((s * drows + kk) * CH, CH)], drain_v)

            @pl.when(c == 0)
            def _():
                pltpu.sync_copy(drain_v, p0_hbm.at[s * drows + kk])

            @pl.when(c == 1)
            def _():
                pltpu.sync_copy(drain_v, p1_hbm.at[s * drows + kk])

    return edge_kernel


def kernel(x, edge_index, edge_attr, W_f, b_f, W_rel, W_root, b):
    n, d = x.shape
    h_dim = W_f.shape[1]
    n_edges = edge_index.shape[1]
    n_rel = edge_attr.shape[1]
    n_layers = W_rel.shape[0]
    n4 = n_rel * n
    eb = n_edges // CH
    half = n // NC
    bn = 1000
    nb = n // bn
    be = 6400
    f32 = jnp.float32

    src = edge_index[0].reshape(1, n_edges)
    dst = edge_index[1].reshape(1, n_edges)
    ea_t = edge_attr.T

    proj = pl.pallas_call(
        _proj_body,
        grid=(nb,),
        in_specs=[
            pl.BlockSpec((bn, d), lambda i: (i, 0)),
            pl.BlockSpec((d, h_dim), lambda i: (0, 0)),
            pl.BlockSpec((1, h_dim), lambda i: (0, 0)),
        ],
        out_specs=pl.BlockSpec((bn, h_dim), lambda i: (i, 0)),
        out_shape=jax.ShapeDtypeStruct((n, h_dim), f32),
    )
    h = proj(x, W_f, b_f.reshape(1, h_dim))

    eprep = pl.pallas_call(
        functools.partial(_eprep_body, n),
        grid=(n_edges // be,),
        in_specs=[
            pl.BlockSpec((n_rel, be), lambda i: (0, i)),
            pl.BlockSpec((1, be), lambda i: (0, i)),
            pl.BlockSpec((1, be), lambda i: (0, i)),
        ],
        out_specs=[
            pl.BlockSpec((1, be), lambda i: (0, i)),
            pl.BlockSpec((1, be), lambda i: (0, i)),
        ],
        out_shape=[jax.ShapeDtypeStruct((1, n_edges), jnp.int32),
                   jax.ShapeDtypeStruct((1, n_edges), jnp.int32)],
    )
    g2, k2 = eprep(ea_t, src, dst)
    k16 = k2.reshape(NS, eb // NS, CH)
    k32 = k2.reshape(NW, eb // NW, CH)
    g32 = g2.reshape(NW, n_edges // NW)
    d32 = dst.reshape(NW, n_edges // NW)

    cnt = _make_hist_kernel(n4, eb)(k16).reshape(n4, LANES)
    w = _make_w_kernel(n4, eb, n_edges)(k32, cnt)
    w32 = w.reshape(NW, n_edges // NW)

    og, od, ow, nblk = _make_partition_kernel(n, n_edges)(g32, d32, w32)

    transform = pl.pallas_call(
        _transform_body,
        grid=(nb, n_rel),
        in_specs=[
            pl.BlockSpec((bn, h_dim), lambda i, r: (i, 0)),
            pl.BlockSpec((1, h_dim, h_dim), lambda i, r: (r, 0, 0)),
            pl.BlockSpec((h_dim, h_dim), lambda i, r: (0, 0)),
            pl.BlockSpec((1, h_dim), lambda i, r: (0, 0)),
        ],
        out_specs=[
            pl.BlockSpec((bn, h_dim), lambda i, r: (r * nb + i, 0)),
            pl.BlockSpec((bn, h_dim), lambda i, r: (i, 0)),
        ],
        out_shape=[jax.ShapeDtypeStruct((n4, h_dim), f32),
                   jax.ShapeDtypeStruct((n, h_dim), f32)],
    )

    combine = pl.pallas_call(
        _combine_body,
        grid=(nb,),
        in_specs=[pl.BlockSpec((bn, h_dim), lambda i: (i, 0))] * 2,
        out_specs=pl.BlockSpec((bn, h_dim), lambda i: (i, 0)),
        out_shape=jax.ShapeDtypeStruct((n, h_dim), f32),
    )

    edge_pass = _make_edge_kernel(n, h_dim)

    for l in range(n_layers):
        h4, base = transform(h, W_rel[l], W_root[l], b[l].reshape(1, h_dim))
        p0, p1 = edge_pass(h4, og, od, ow, nblk)
        p = jnp.concatenate(
            [p0.reshape(-1, h_dim)[:half], p1.reshape(-1, h_dim)[:half]], 0)
        h = combine(base, p)
    return h


# speedup vs baseline: 2.5096x; 1.0275x over previous
import functools

import jax
import jax.numpy as jnp
from jax import lax
from jax.experimental import pallas as pl
from jax.experimental.pallas import tpu as pltpu
from jax.experimental.pallas import tpu_sc as plsc

NC = 2
NS = 16
NW = NC * NS
LANES = 16
CH = 80
BLK = 800
NBLK_CAP = 14
REG_CAP = NBLK_CAP * BLK

_MESH = dict(core_axis_name="c", subcore_axis_name="s")
_SC_PARAMS = pltpu.CompilerParams(
    use_tc_tiling_on_sc=False, needs_layout_passes=False)


def _proj_body(x_ref, wf_ref, bf_ref, o_ref):
    o_ref[...] = jnp.maximum(
        jnp.dot(x_ref[...], wf_ref[...], preferred_element_type=jnp.float32)
        + bf_ref[...], 0.0)


def _eprep_body(n_nodes, ea_ref, src_ref, dst_ref, g_ref, k_ref):
    ea = ea_ref[...]
    mx = ea[0:1, :]
    et = jnp.zeros(mx.shape, jnp.int32)
    for r in range(1, ea.shape[0]):
        row = ea[r:r + 1, :]
        gt = row > mx
        et = jnp.where(gt, r, et)
        mx = jnp.where(gt, row, mx)
    g_ref[...] = et * n_nodes + src_ref[...]
    k_ref[...] = et * n_nodes + dst_ref[...]


def _transform_body(h_ref, wr_ref, wroot_ref, bl_ref, h4_ref, base_ref):
    r = pl.program_id(1)
    hv = h_ref[...]

    @pl.when(r == 0)
    def _():
        base_ref[...] = jnp.dot(
            hv, wroot_ref[...], preferred_element_type=jnp.float32) + bl_ref[...]

    h4_ref[...] = jnp.dot(hv, wr_ref[0], preferred_element_type=jnp.float32)


def _combine_body(b_ref, p_ref, o_ref):
    o_ref[...] = jnp.maximum(b_ref[...] + p_ref[...], 0.0)


def _make_hist_kernel(n4, eb):
    rows_per_tec = eb // NS
    zrows = n4 // NW

    @functools.partial(
        pl.kernel,
        out_type=jax.ShapeDtypeStruct((NW, zrows, LANES), jnp.float32),
        mesh=plsc.VectorSubcoreMesh(**_MESH),
        compiler_params=_SC_PARAMS,
        scratch_types=[
            pltpu.VMEM((rows_per_tec, CH), jnp.int32),
            pltpu.VMEM((CH, LANES), jnp.float32),
            pltpu.VMEM((zrows, LANES), jnp.float32),
            pltpu.VMEM_SHARED((n4, LANES), jnp.float32),
        ],
    )
    def hist_kernel(key_hbm, cnt_hbm, kidx_v, ones_v, dz_v, acc):
        c = lax.axis_index("c")
        s = lax.axis_index("s")

        def initr(i, _):
            ones_v[i, :] = jnp.ones((LANES,), jnp.float32)
            return 0
        lax.fori_loop(0, CH, initr, 0)

        def zr(i, _):
            dz_v[i, :] = jnp.zeros((LANES,), jnp.float32)
            return 0
        lax.fori_loop(0, zrows, zr, 0)

        per_tec = n4 // NS
        for kk in range(per_tec // zrows):
            pltpu.sync_copy(dz_v, acc.at[pl.ds(s * per_tec + kk * zrows, zrows)])
        plsc.subcore_barrier()

        pltpu.sync_copy(key_hbm.at[s], kidx_v)

        def chunk(i, _):
            pltpu.sync_copy(ones_v, acc.at[kidx_v.at[i]], add=True)
            return 0
        lax.fori_loop(0, rows_per_tec, chunk, 0)
        plsc.subcore_barrier()

        dd = c * NS + s
        pltpu.sync_copy(acc.at[pl.ds(dd * zrows, zrows)], dz_v)
        pltpu.sync_copy(dz_v, cnt_hbm.at[dd])

    return hist_kernel


def _make_w_kernel(n4, eb, n_edges):
    rows_per_w = eb // NW

    @functools.partial(
        pl.kernel,
        out_type=jax.ShapeDtypeStruct((n_edges,), jnp.float32),
        mesh=plsc.VectorSubcoreMesh(**_MESH),
        compiler_params=_SC_PARAMS,
        scratch_types=[
            pltpu.VMEM((rows_per_w, CH), jnp.int32),
            pltpu.VMEM((CH, LANES), jnp.float32),
            pltpu.VMEM((CH,), jnp.float32),
            pltpu.SemaphoreType.DMA,
        ],
    )
    def w_kernel(key_hbm, cnt_hbm, w_hbm, kidx_v, crows_v, wbuf_v, sem):
        c = lax.axis_index("c")
        s = lax.axis_index("s")
        wid = s * NC + c
        pltpu.sync_copy(key_hbm.at[wid], kidx_v)
        lane = lax.iota(jnp.int32, LANES)
        zeros16 = jnp.zeros((LANES,), jnp.int32)

        def chunk(i, _):
            pltpu.async_copy(cnt_hbm.at[kidx_v.at[i]], crows_v, sem).wait()

            def sub(j, _):
                ridx = j * LANES + lane
                c16 = plsc.load_gather(crows_v, [ridx, zeros16])
                wbuf_v[pl.ds(j * LANES, LANES)] = 1.0 / jnp.maximum(c16, 1.0)
                return 0
            lax.fori_loop(0, CH // LANES, sub, 0)
            pltpu.sync_copy(
                wbuf_v, w_hbm.at[pl.ds((wid * rows_per_w + i) * CH, CH)])
            return 0
        lax.fori_loop(0, rows_per_w, chunk, 0)

    return w_kernel


def _make_partition_kernel(n_nodes, n_edges):
    e_per_w = n_edges // NW
    half = n_nodes // NC
    groups = e_per_w // LANES

    @functools.partial(
        pl.kernel,
        out_type=(jax.ShapeDtypeStruct((NC, NW, REG_CAP), jnp.int32),
                  jax.ShapeDtypeStruct((NC, NW, REG_CAP), jnp.int32),
                  jax.ShapeDtypeStruct((NC, NW, REG_CAP), jnp.float32),
                  jax.ShapeDtypeStruct((NC, NW, LANES), jnp.int32)),
        mesh=plsc.VectorSubcoreMesh(**_MESH),
        compiler_params=_SC_PARAMS,
        scratch_types=[
            pltpu.VMEM((e_per_w,), jnp.int32),
            pltpu.VMEM((e_per_w,), jnp.int32),
            pltpu.VMEM((e_per_w,), jnp.float32),
            pltpu.VMEM((NC, REG_CAP), jnp.int32),
            pltpu.VMEM((NC, REG_CAP), jnp.int32),
            pltpu.VMEM((NC, REG_CAP), jnp.float32),
            pltpu.VMEM((LANES,), jnp.int32),
        ],
    )
    def part_kernel(g_hbm, d_hbm, w_hbm,
                    og_hbm, od_hbm, ow_hbm, nb_hbm,
                    gb_v, db_v, wb_v, og_v, od_v, ow_v, nb_v):
        c = lax.axis_index("c")
        s = lax.axis_index("s")
        wid = s * NC + c

        pltpu.sync_copy(g_hbm.at[wid], gb_v)
        pltpu.sync_copy(d_hbm.at[wid], db_v)
        pltpu.sync_copy(w_hbm.at[wid], wb_v)

        def group(k, offs):
            o0, o1 = offs
            g16 = gb_v[pl.ds(k * LANES, LANES)]
            d16 = db_v[pl.ds(k * LANES, LANES)]
            w16 = wb_v[pl.ds(k * LANES, LANES)]
            m0 = d16 < half
            m1 = jnp.logical_not(m0)
            plsc.store_compressed(og_v.at[0, pl.ds(o0, LANES)], g16, mask=m0)
            plsc.store_compressed(od_v.at[0, pl.ds(o0, LANES)], d16, mask=m0)
            plsc.store_compressed(ow_v.at[0, pl.ds(o0, LANES)], w16, mask=m0)
            n0 = lax.reduce_max(
                plsc.all_reduce_population_count(m0), axes=(0,))
            plsc.store_compressed(og_v.at[1, pl.ds(o1, LANES)], g16, mask=m1)
            plsc.store_compressed(od_v.at[1, pl.ds(o1, LANES)], d16 - half, mask=m1)
            plsc.store_compressed(ow_v.at[1, pl.ds(o1, LANES)], w16, mask=m1)
            n1 = lax.reduce_max(
                plsc.all_reduce_population_count(m1), axes=(0,))
            return (o0 + n0, o1 + n1)

        o0, o1 = lax.fori_loop(0, groups, group, (jnp.int32(0), jnp.int32(0)))

        zi16 = jnp.zeros((LANES,), jnp.int32)
        dummyd = jnp.full((LANES,), half, jnp.int32)
        zf16 = jnp.zeros((LANES,), jnp.float32)
        for hh, off in ((0, o0), (1, o1)):
            for i in range(BLK // LANES):
                og_v[hh, pl.ds(off + i * LANES, LANES)] = zi16
                od_v[hh, pl.ds(off + i * LANES, LANES)] = dummyd
                ow_v[hh, pl.ds(off + i * LANES, LANES)] = zf16

        pltpu.sync_copy(og_v.at[0], og_hbm.at[0, wid])
        pltpu.sync_copy(od_v.at[0], od_hbm.at[0, wid])
        pltpu.sync_copy(ow_v.at[0], ow_hbm.at[0, wid])
        pltpu.sync_copy(og_v.at[1], og_hbm.at[1, wid])
        pltpu.sync_copy(od_v.at[1], od_hbm.at[1, wid])
        pltpu.sync_copy(ow_v.at[1], ow_hbm.at[1, wid])

        for hh, off in ((0, o0), (1, o1)):
            nblk = (off + (BLK - 1)) // BLK
            nb_v[:] = jnp.broadcast_to(nblk, (LANES,)).astype(jnp.int32)
            pltpu.sync_copy(nb_v, nb_hbm.at[hh, wid])

    return part_kernel


def _make_edge_kernel(n_nodes, h_dim):
    half = n_nodes // NC
    acc_rows = ((half // (NS * CH)) + 1) * NS * CH
    nsub = h_dim // LANES
    drows = acc_rows // NS // CH

    @functools.partial(
        pl.kernel,
        out_type=(jax.ShapeDtypeStruct((NS * drows, CH, h_dim), jnp.float32),
                  jax.ShapeDtypeStruct((NS * drows, CH, h_dim), jnp.float32)),
        mesh=plsc.VectorSubcoreMesh(**_MESH),
        compiler_params=_SC_PARAMS,
        scratch_types=[
            pltpu.VMEM((BLK,), jnp.int32),
            pltpu.VMEM((BLK,), jnp.int32),
            pltpu.VMEM((BLK // CH, CH), jnp.int32),
            pltpu.VMEM((BLK,), jnp.float32),
            pltpu.VMEM((LANES,), jnp.int32),
            pltpu.VMEM((CH, h_dim), jnp.float32),
            pltpu.VMEM((CH, h_dim), jnp.float32),
            pltpu.VMEM((CH, h_dim), jnp.float32),
            pltpu.VMEM((CH, h_dim), jnp.float32),
            pltpu.SemaphoreType.DMA,
            pltpu.SemaphoreType.DMA,
            pltpu.VMEM_SHARED((acc_rows, h_dim), jnp.float32),
        ],
    )
    def edge_kernel(h4_hbm, og_hbm, od_hbm, ow_hbm, nb_hbm, p0_hbm, p1_hbm,
                    gf_v, df_v, di_v, w_v, nb_v, rows0_v, rows1_v, rows2_v,
                    rows3_v, semg, sems, acc):
        c = lax.axis_index("c")
        s = lax.axis_index("s")
        nch = BLK // CH
        rows = (rows0_v, rows1_v, rows2_v, rows3_v)
        nbuf = len(rows)

        def zr(i, _):
            for t in range(nsub):
                rows0_v[i, pl.ds(t * LANES, LANES)] = jnp.zeros(
                    (LANES,), jnp.float32)
            return 0
        lax.fori_loop(0, CH, zr, 0)
        for kk in range(drows):
            pltpu.sync_copy(
                rows0_v, acc.at[pl.ds((s * drows + kk) * CH, CH)])
        plsc.subcore_barrier()

        for ridx in range(NW // NS):
            rg = s + NS * ridx
            pltpu.sync_copy(nb_hbm.at[c, rg], nb_v)
            nblk = lax.reduce_max(nb_v[:], axes=(0,))

            def block(bi, _):
                base = bi * BLK
                pltpu.sync_copy(og_hbm.at[c, rg, pl.ds(base, BLK)], gf_v)
                pltpu.sync_copy(od_hbm.at[c, rg, pl.ds(base, BLK)], df_v)
                pltpu.sync_copy(ow_hbm.at[c, rg, pl.ds(base, BLK)], w_v)
                for i in range(nch):
                    for j in range(CH // LANES):
                        di_v[i, pl.ds(j * LANES, LANES)] = (
                            df_v[pl.ds(i * CH + j * LANES, LANES)])

                descs_g = [None] * nch
                descs_s = [None] * nch

                def gather(k):
                    return pltpu.async_copy(
                        h4_hbm.at[gf_v.at[pl.ds(k * CH, CH)]],
                        rows[k % nbuf], semg)

                for k in range(min(nbuf - 1, nch)):
                    descs_g[k] = gather(k)
                for k in range(nch):
                    descs_g[k].wait()
                    rbuf = rows[k % nbuf]

                    def scale(j2, _, k=k, rbuf=rbuf):
                        wv = plsc.load_gather(
                            w_v, [jnp.full((LANES,), k * CH + j2, jnp.int32)])
                        for t in range(nsub):
                            rbuf[j2, pl.ds(t * LANES, LANES)] = (
                                rbuf[j2, pl.ds(t * LANES, LANES)] * wv)
                        return 0
                    lax.fori_loop(0, CH, scale, 0)
                    descs_s[k] = pltpu.async_copy(
                        rbuf, acc.at[di_v.at[k]], sems, add=True)
                    nk = k + nbuf - 1
                    if nk < nch:
                        if nk - nbuf >= 0:
                            descs_s[nk - nbuf].wait()
                        descs_g[nk] = gather(nk)
                for k in range(max(0, nch - nbuf), nch):
                    descs_s[k].wait()
                return 0
            lax.fori_loop(0, nblk, block, 0)

        plsc.subcore_barrier()
        for kk in range(drows):
            pltpu.sync_copy(acc.at[pl.ds((s * drows + kk) * CH, CH)], rows0_v)

            @pl.when(c == 0)
            def _():
                pltpu.sync_copy(rows0_v, p0_hbm.at[s * drows + kk])

            @pl.when(c == 1)
            def _():
                pltpu.sync_copy(rows0_v, p1_hbm.at[s * drows + kk])

    return edge_kernel


def kernel(x, edge_index, edge_attr, W_f, b_f, W_rel, W_root, b):
    n, d = x.shape
    h_dim = W_f.shape[1]
    n_edges = edge_index.shape[1]
    n_rel = edge_attr.shape[1]
    n_layers = W_rel.shape[0]
    n4 = n_rel * n
    eb = n_edges // CH
    half = n // NC
    bn = 1000
    nb = n // bn
    be = 6400
    f32 = jnp.float32

    src = edge_index[0].reshape(1, n_edges)
    dst = edge_index[1].reshape(1, n_edges)
    ea_t = edge_attr.T

    proj = pl.pallas_call(
        _proj_body,
        grid=(nb,),
        in_specs=[
            pl.BlockSpec((bn, d), lambda i: (i, 0)),
            pl.BlockSpec((d, h_dim), lambda i: (0, 0)),
            pl.BlockSpec((1, h_dim), lambda i: (0, 0)),
        ],
        out_specs=pl.BlockSpec((bn, h_dim), lambda i: (i, 0)),
        out_shape=jax.ShapeDtypeStruct((n, h_dim), f32),
    )
    h = proj(x, W_f, b_f.reshape(1, h_dim))

    eprep = pl.pallas_call(
        functools.partial(_eprep_body, n),
        grid=(n_edges // be,),
        in_specs=[
            pl.BlockSpec((n_rel, be), lambda i: (0, i)),
            pl.BlockSpec((1, be), lambda i: (0, i)),
            pl.BlockSpec((1, be), lambda i: (0, i)),
        ],
        out_specs=[
            pl.BlockSpec((1, be), lambda i: (0, i)),
            pl.BlockSpec((1, be), lambda i: (0, i)),
        ],
        out_shape=[jax.ShapeDtypeStruct((1, n_edges), jnp.int32),
                   jax.ShapeDtypeStruct((1, n_edges), jnp.int32)],
    )
    g2, k2 = eprep(ea_t, src, dst)
    k16 = k2.reshape(NS, eb // NS, CH)
    k32 = k2.reshape(NW, eb // NW, CH)
    g32 = g2.reshape(NW, n_edges // NW)
    d32 = dst.reshape(NW, n_edges // NW)

    cnt = _make_hist_kernel(n4, eb)(k16).reshape(n4, LANES)
    w = _make_w_kernel(n4, eb, n_edges)(k32, cnt)
    w32 = w.reshape(NW, n_edges // NW)

    og, od, ow, nblk = _make_partition_kernel(n, n_edges)(g32, d32, w32)

    transform = pl.pallas_call(
        _transform_body,
        grid=(nb, n_rel),
        in_specs=[
            pl.BlockSpec((bn, h_dim), lambda i, r: (i, 0)),
            pl.BlockSpec((1, h_dim, h_dim), lambda i, r: (r, 0, 0)),
            pl.BlockSpec((h_dim, h_dim), lambda i, r: (0, 0)),
            pl.BlockSpec((1, h_dim), lambda i, r: (0, 0)),
        ],
        out_specs=[
            pl.BlockSpec((bn, h_dim), lambda i, r: (r * nb + i, 0)),
            pl.BlockSpec((bn, h_dim), lambda i, r: (i, 0)),
        ],
        out_shape=[jax.ShapeDtypeStruct((n4, h_dim), f32),
                   jax.ShapeDtypeStruct((n, h_dim), f32)],
    )

    combine = pl.pallas_call(
        _combine_body,
        grid=(nb,),
        in_specs=[pl.BlockSpec((bn, h_dim), lambda i: (i, 0))] * 2,
        out_specs=pl.BlockSpec((bn, h_dim), lambda i: (i, 0)),
        out_shape=jax.ShapeDtypeStruct((n, h_dim), f32),
    )

    edge_pass = _make_edge_kernel(n, h_dim)

    for l in range(n_layers):
        h4, base = transform(h, W_rel[l], W_root[l], b[l].reshape(1, h_dim))
        p0, p1 = edge_pass(h4, og, od, ow, nblk)
        p = jnp.concatenate(
            [p0.reshape(-1, h_dim)[:half], p1.reshape(-1, h_dim)[:half]], 0)
        h = combine(base, p)
    return h


# speedup vs baseline: 3.7948x; 1.5121x over previous
import functools

import jax
import jax.numpy as jnp
from jax import lax
from jax.experimental import pallas as pl
from jax.experimental.pallas import tpu as pltpu
from jax.experimental.pallas import tpu_sc as plsc

NC = 2
NS = 16
NW = NC * NS
LANES = 16
CH = 80
BLK = 800
NBLK_CAP = 14
REG_CAP = NBLK_CAP * BLK

_MESH = dict(core_axis_name="c", subcore_axis_name="s")
_SC_PARAMS = pltpu.CompilerParams(
    use_tc_tiling_on_sc=False, needs_layout_passes=False)


def _proj_body(x_ref, wf_ref, bf_ref, o_ref):
    o_ref[...] = jnp.maximum(
        jnp.dot(x_ref[...], wf_ref[...], preferred_element_type=jnp.float32)
        + bf_ref[...], 0.0)


def _eprep_body(n_nodes, ea_ref, src_ref, dst_ref, g_ref, k_ref):
    ea = ea_ref[...]
    mx = ea[0:1, :]
    et = jnp.zeros(mx.shape, jnp.int32)
    for r in range(1, ea.shape[0]):
        row = ea[r:r + 1, :]
        gt = row > mx
        et = jnp.where(gt, r, et)
        mx = jnp.where(gt, row, mx)
    g_ref[...] = et * n_nodes + src_ref[...]
    k_ref[...] = et * n_nodes + dst_ref[...]


def _transform_body(h_ref, wr_ref, wroot_ref, bl_ref, pm_ref, h4_ref, base_ref):
    r = pl.program_id(1)
    hv = h_ref[...]

    @pl.when(r == 0)
    def _():
        base_ref[...] = jnp.dot(
            hv, wroot_ref[...], preferred_element_type=jnp.float32) + bl_ref[...]

    hr = jnp.dot(hv, wr_ref[0], preferred_element_type=jnp.float32)
    h4_ref[...] = jnp.dot(
        hr, pm_ref[...], preferred_element_type=jnp.float32).astype(jnp.bfloat16)


def _combine_body(b_ref, p_ref, o_ref):
    o_ref[...] = jnp.maximum(b_ref[...] + p_ref[...], 0.0)


def _make_hist_kernel(n4, eb):
    rows_per_tec = eb // NS
    zrows = n4 // NW

    @functools.partial(
        pl.kernel,
        out_type=jax.ShapeDtypeStruct((NW, zrows, LANES), jnp.float32),
        mesh=plsc.VectorSubcoreMesh(**_MESH),
        compiler_params=_SC_PARAMS,
        scratch_types=[
            pltpu.VMEM((rows_per_tec, CH), jnp.int32),
            pltpu.VMEM((CH, LANES), jnp.float32),
            pltpu.VMEM((zrows, LANES), jnp.float32),
            pltpu.VMEM_SHARED((n4, LANES), jnp.float32),
        ],
    )
    def hist_kernel(key_hbm, cnt_hbm, kidx_v, ones_v, dz_v, acc):
        c = lax.axis_index("c")
        s = lax.axis_index("s")

        def initr(i, _):
            ones_v[i, :] = jnp.ones((LANES,), jnp.float32)
            return 0
        lax.fori_loop(0, CH, initr, 0)

        def zr(i, _):
            dz_v[i, :] = jnp.zeros((LANES,), jnp.float32)
            return 0
        lax.fori_loop(0, zrows, zr, 0)

        per_tec = n4 // NS
        for kk in range(per_tec // zrows):
            pltpu.sync_copy(dz_v, acc.at[pl.ds(s * per_tec + kk * zrows, zrows)])
        plsc.subcore_barrier()

        pltpu.sync_copy(key_hbm.at[s], kidx_v)

        def chunk(i, _):
            pltpu.sync_copy(ones_v, acc.at[kidx_v.at[i]], add=True)
            return 0
        lax.fori_loop(0, rows_per_tec, chunk, 0)
        plsc.subcore_barrier()

        dd = c * NS + s
        pltpu.sync_copy(acc.at[pl.ds(dd * zrows, zrows)], dz_v)
        pltpu.sync_copy(dz_v, cnt_hbm.at[dd])

    return hist_kernel


def _make_w_kernel(n4, eb, n_edges):
    rows_per_w = eb // NW

    @functools.partial(
        pl.kernel,
        out_type=jax.ShapeDtypeStruct((n_edges,), jnp.float32),
        mesh=plsc.VectorSubcoreMesh(**_MESH),
        compiler_params=_SC_PARAMS,
        scratch_types=[
            pltpu.VMEM((rows_per_w, CH), jnp.int32),
            pltpu.VMEM((CH, LANES), jnp.float32),
            pltpu.VMEM((CH,), jnp.float32),
            pltpu.SemaphoreType.DMA,
        ],
    )
    def w_kernel(key_hbm, cnt_hbm, w_hbm, kidx_v, crows_v, wbuf_v, sem):
        c = lax.axis_index("c")
        s = lax.axis_index("s")
        wid = s * NC + c
        pltpu.sync_copy(key_hbm.at[wid], kidx_v)
        lane = lax.iota(jnp.int32, LANES)
        zeros16 = jnp.zeros((LANES,), jnp.int32)

        def chunk(i, _):
            pltpu.async_copy(cnt_hbm.at[kidx_v.at[i]], crows_v, sem).wait()

            def sub(j, _):
                ridx = j * LANES + lane
                c16 = plsc.load_gather(crows_v, [ridx, zeros16])
                wbuf_v[pl.ds(j * LANES, LANES)] = 1.0 / jnp.maximum(c16, 1.0)
                return 0
            lax.fori_loop(0, CH // LANES, sub, 0)
            pltpu.sync_copy(
                wbuf_v, w_hbm.at[pl.ds((wid * rows_per_w + i) * CH, CH)])
            return 0
        lax.fori_loop(0, rows_per_w, chunk, 0)

    return w_kernel


def _make_partition_kernel(n_nodes, n_edges):
    e_per_w = n_edges // NW
    half = n_nodes // NC
    groups = e_per_w // LANES

    @functools.partial(
        pl.kernel,
        out_type=(jax.ShapeDtypeStruct((NC, NW, REG_CAP), jnp.int32),
                  jax.ShapeDtypeStruct((NC, NW, REG_CAP), jnp.int32),
                  jax.ShapeDtypeStruct((NC, NW, REG_CAP), jnp.float32),
                  jax.ShapeDtypeStruct((NC, NW, LANES), jnp.int32)),
        mesh=plsc.VectorSubcoreMesh(**_MESH),
        compiler_params=_SC_PARAMS,
        scratch_types=[
            pltpu.VMEM((e_per_w,), jnp.int32),
            pltpu.VMEM((e_per_w,), jnp.int32),
            pltpu.VMEM((e_per_w,), jnp.float32),
            pltpu.VMEM((NC, REG_CAP), jnp.int32),
            pltpu.VMEM((NC, REG_CAP), jnp.int32),
            pltpu.VMEM((NC, REG_CAP), jnp.float32),
            pltpu.VMEM((LANES,), jnp.int32),
        ],
    )
    def part_kernel(g_hbm, d_hbm, w_hbm,
                    og_hbm, od_hbm, ow_hbm, nb_hbm,
                    gb_v, db_v, wb_v, og_v, od_v, ow_v, nb_v):
        c = lax.axis_index("c")
        s = lax.axis_index("s")
        wid = s * NC + c

        pltpu.sync_copy(g_hbm.at[wid], gb_v)
        pltpu.sync_copy(d_hbm.at[wid], db_v)
        pltpu.sync_copy(w_hbm.at[wid], wb_v)

        def group(k, offs):
            o0, o1 = offs
            g16 = gb_v[pl.ds(k * LANES, LANES)]
            d16 = db_v[pl.ds(k * LANES, LANES)]
            w16 = wb_v[pl.ds(k * LANES, LANES)]
            m0 = d16 < half
            m1 = jnp.logical_not(m0)
            plsc.store_compressed(og_v.at[0, pl.ds(o0, LANES)], g16, mask=m0)
            plsc.store_compressed(od_v.at[0, pl.ds(o0, LANES)], d16, mask=m0)
            plsc.store_compressed(ow_v.at[0, pl.ds(o0, LANES)], w16, mask=m0)
            n0 = lax.reduce_max(
                plsc.all_reduce_population_count(m0), axes=(0,))
            plsc.store_compressed(og_v.at[1, pl.ds(o1, LANES)], g16, mask=m1)
            plsc.store_compressed(od_v.at[1, pl.ds(o1, LANES)], d16 - half, mask=m1)
            plsc.store_compressed(ow_v.at[1, pl.ds(o1, LANES)], w16, mask=m1)
            n1 = lax.reduce_max(
                plsc.all_reduce_population_count(m1), axes=(0,))
            return (o0 + n0, o1 + n1)

        o0, o1 = lax.fori_loop(0, groups, group, (jnp.int32(0), jnp.int32(0)))

        zi16 = jnp.zeros((LANES,), jnp.int32)
        dummyd = jnp.full((LANES,), half, jnp.int32)
        zf16 = jnp.zeros((LANES,), jnp.float32)
        for hh, off in ((0, o0), (1, o1)):
            for i in range(BLK // LANES):
                og_v[hh, pl.ds(off + i * LANES, LANES)] = zi16
                od_v[hh, pl.ds(off + i * LANES, LANES)] = dummyd
                ow_v[hh, pl.ds(off + i * LANES, LANES)] = zf16

        pltpu.sync_copy(og_v.at[0], og_hbm.at[0, wid])
        pltpu.sync_copy(od_v.at[0], od_hbm.at[0, wid])
        pltpu.sync_copy(ow_v.at[0], ow_hbm.at[0, wid])
        pltpu.sync_copy(og_v.at[1], og_hbm.at[1, wid])
        pltpu.sync_copy(od_v.at[1], od_hbm.at[1, wid])
        pltpu.sync_copy(ow_v.at[1], ow_hbm.at[1, wid])

        for hh, off in ((0, o0), (1, o1)):
            nblk = (off + (BLK - 1)) // BLK
            nb_v[:] = jnp.broadcast_to(nblk, (LANES,)).astype(jnp.int32)
            pltpu.sync_copy(nb_v, nb_hbm.at[hh, wid])

    return part_kernel


def _make_edge_kernel(n_nodes, h_dim):
    half = n_nodes // NC
    acc_rows = ((half // (NS * CH)) + 1) * NS * CH
    nsub = h_dim // LANES
    drows = acc_rows // NS // CH

    @functools.partial(
        pl.kernel,
        out_type=(jax.ShapeDtypeStruct((NS * drows, CH, h_dim), jnp.float32),
                  jax.ShapeDtypeStruct((NS * drows, CH, h_dim), jnp.float32)),
        mesh=plsc.VectorSubcoreMesh(**_MESH),
        compiler_params=_SC_PARAMS,
        scratch_types=[
            pltpu.VMEM((BLK,), jnp.int32),
            pltpu.VMEM((BLK,), jnp.int32),
            pltpu.VMEM((BLK // CH, CH), jnp.int32),
            pltpu.VMEM((BLK,), jnp.float32),
            pltpu.VMEM((LANES,), jnp.int32),
            pltpu.VMEM((CH, h_dim), jnp.bfloat16),
            pltpu.VMEM((CH, h_dim), jnp.bfloat16),
            pltpu.VMEM((CH, h_dim), jnp.bfloat16),
            pltpu.VMEM((CH, h_dim), jnp.bfloat16),
            pltpu.VMEM((CH, h_dim), jnp.float32),
            pltpu.VMEM((CH, h_dim), jnp.float32),
            pltpu.SemaphoreType.DMA,
            pltpu.SemaphoreType.DMA,
            pltpu.VMEM_SHARED((acc_rows, h_dim), jnp.float32),
        ],
    )
    def edge_kernel(h4_hbm, og_hbm, od_hbm, ow_hbm, nb_hbm, p0_hbm, p1_hbm,
                    gf_v, df_v, di_v, w_v, nb_v, rows0_v, rows1_v, rows2_v,
                    rows3_v, sc0_v, sc1_v, semg, sems, acc):
        c = lax.axis_index("c")
        s = lax.axis_index("s")
        nch = BLK // CH
        rows = (rows0_v, rows1_v, rows2_v, rows3_v)
        sbufs = (sc0_v, sc1_v)
        nbuf = len(rows)

        def zr(i, _):
            for t in range(nsub):
                sc0_v[i, pl.ds(t * LANES, LANES)] = jnp.zeros(
                    (LANES,), jnp.float32)
            return 0
        lax.fori_loop(0, CH, zr, 0)
        for kk in range(drows):
            pltpu.sync_copy(
                sc0_v, acc.at[pl.ds((s * drows + kk) * CH, CH)])
        plsc.subcore_barrier()

        for ridx in range(NW // NS):
            rg = s + NS * ridx
            pltpu.sync_copy(nb_hbm.at[c, rg], nb_v)
            nblk = lax.reduce_max(nb_v[:], axes=(0,))

            def block(bi, _):
                base = bi * BLK
                pltpu.sync_copy(og_hbm.at[c, rg, pl.ds(base, BLK)], gf_v)
                pltpu.sync_copy(od_hbm.at[c, rg, pl.ds(base, BLK)], df_v)
                pltpu.sync_copy(ow_hbm.at[c, rg, pl.ds(base, BLK)], w_v)
                for i in range(nch):
                    for j in range(CH // LANES):
                        di_v[i, pl.ds(j * LANES, LANES)] = (
                            df_v[pl.ds(i * CH + j * LANES, LANES)])

                descs_g = [None] * nch
                descs_s = [None] * nch

                def gather(k):
                    return pltpu.async_copy(
                        h4_hbm.at[gf_v.at[pl.ds(k * CH, CH)]],
                        rows[k % nbuf], semg)

                for k in range(min(nbuf - 1, nch)):
                    descs_g[k] = gather(k)
                for k in range(nch):
                    descs_g[k].wait()
                    if k + nbuf - 1 < nch:
                        descs_g[k + nbuf - 1] = gather(k + nbuf - 1)
                    if k - 2 >= 0:
                        descs_s[k - 2].wait()
                    rbuf = rows[k % nbuf]
                    sbuf = sbufs[k % 2]

                    def scale(j2, _, k=k, rbuf=rbuf, sbuf=sbuf):
                        wv = plsc.load_gather(
                            w_v, [jnp.full((LANES,), k * CH + j2, jnp.int32)])
                        for t in range(nsub // 2):
                            v32 = rbuf[j2, pl.ds(t * 2 * LANES, 2 * LANES)]
                            a, bb = plsc.unpack(
                                v32, format=plsc.PackFormat.INTERLEAVED)
                            sbuf[j2, pl.ds(t * 2 * LANES, LANES)] = a * wv
                            sbuf[j2, pl.ds(t * 2 * LANES + LANES, LANES)] = (
                                bb * wv)
                        return 0
                    lax.fori_loop(0, CH, scale, 0)
                    descs_s[k] = pltpu.async_copy(
                        sbuf, acc.at[di_v.at[k]], sems, add=True)
                for k in range(max(0, nch - 2), nch):
                    descs_s[k].wait()
                return 0
            lax.fori_loop(0, nblk, block, 0)

        plsc.subcore_barrier()
        for kk in range(drows):
            pltpu.sync_copy(acc.at[pl.ds((s * drows + kk) * CH, CH)], sc0_v)

            @pl.when(c == 0)
            def _():
                pltpu.sync_copy(sc0_v, p0_hbm.at[s * drows + kk])

            @pl.when(c == 1)
            def _():
                pltpu.sync_copy(sc0_v, p1_hbm.at[s * drows + kk])

    return edge_kernel


def kernel(x, edge_index, edge_attr, W_f, b_f, W_rel, W_root, b):
    n, d = x.shape
    h_dim = W_f.shape[1]
    n_edges = edge_index.shape[1]
    n_rel = edge_attr.shape[1]
    n_layers = W_rel.shape[0]
    n4 = n_rel * n
    eb = n_edges // CH
    half = n // NC
    bn = 1000
    nb = n // bn
    be = 6400
    f32 = jnp.float32

    src = edge_index[0].reshape(1, n_edges)
    dst = edge_index[1].reshape(1, n_edges)
    ea_t = edge_attr.T

    proj = pl.pallas_call(
        _proj_body,
        grid=(nb,),
        in_specs=[
            pl.BlockSpec((bn, d), lambda i: (i, 0)),
            pl.BlockSpec((d, h_dim), lambda i: (0, 0)),
            pl.BlockSpec((1, h_dim), lambda i: (0, 0)),
        ],
        out_specs=pl.BlockSpec((bn, h_dim), lambda i: (i, 0)),
        out_shape=jax.ShapeDtypeStruct((n, h_dim), f32),
    )
    h = proj(x, W_f, b_f.reshape(1, h_dim))

    eprep = pl.pallas_call(
        functools.partial(_eprep_body, n),
        grid=(n_edges // be,),
        in_specs=[
            pl.BlockSpec((n_rel, be), lambda i: (0, i)),
            pl.BlockSpec((1, be), lambda i: (0, i)),
            pl.BlockSpec((1, be), lambda i: (0, i)),
        ],
        out_specs=[
            pl.BlockSpec((1, be), lambda i: (0, i)),
            pl.BlockSpec((1, be), lambda i: (0, i)),
        ],
        out_shape=[jax.ShapeDtypeStruct((1, n_edges), jnp.int32),
                   jax.ShapeDtypeStruct((1, n_edges), jnp.int32)],
    )
    g2, k2 = eprep(ea_t, src, dst)
    k16 = k2.reshape(NS, eb // NS, CH)
    k32 = k2.reshape(NW, eb // NW, CH)
    g32 = g2.reshape(NW, n_edges // NW)
    d32 = dst.reshape(NW, n_edges // NW)

    cnt = _make_hist_kernel(n4, eb)(k16).reshape(n4, LANES)
    w = _make_w_kernel(n4, eb, n_edges)(k32, cnt)
    w32 = w.reshape(NW, n_edges // NW)

    og, od, ow, nblk = _make_partition_kernel(n, n_edges)(g32, d32, w32)

    import numpy as _np
    sigma = _np.empty((h_dim,), _np.int64)
    for t in range(h_dim // 32):
        for i in range(16):
            sigma[32 * t + 2 * i] = 32 * t + i
            sigma[32 * t + 2 * i + 1] = 32 * t + 16 + i
    perm_mat = jnp.asarray(_np.eye(h_dim, dtype=_np.float32)[sigma].T)

    transform = pl.pallas_call(
        _transform_body,
        grid=(nb, n_rel),
        in_specs=[
            pl.BlockSpec((bn, h_dim), lambda i, r: (i, 0)),
            pl.BlockSpec((1, h_dim, h_dim), lambda i, r: (r, 0, 0)),
            pl.BlockSpec((h_dim, h_dim), lambda i, r: (0, 0)),
            pl.BlockSpec((1, h_dim), lambda i, r: (0, 0)),
            pl.BlockSpec((h_dim, h_dim), lambda i, r: (0, 0)),
        ],
        out_specs=[
            pl.BlockSpec((bn, h_dim), lambda i, r: (r * nb + i, 0)),
            pl.BlockSpec((bn, h_dim), lambda i, r: (i, 0)),
        ],
        out_shape=[jax.ShapeDtypeStruct((n4, h_dim), jnp.bfloat16),
                   jax.ShapeDtypeStruct((n, h_dim), f32)],
    )

    combine = pl.pallas_call(
        _combine_body,
        grid=(nb,),
        in_specs=[pl.BlockSpec((bn, h_dim), lambda i: (i, 0))] * 2,
        out_specs=pl.BlockSpec((bn, h_dim), lambda i: (i, 0)),
        out_shape=jax.ShapeDtypeStruct((n, h_dim), f32),
    )

    edge_pass = _make_edge_kernel(n, h_dim)

    for l in range(n_layers):
        h4, base = transform(
            h, W_rel[l], W_root[l], b[l].reshape(1, h_dim), perm_mat)
        p0, p1 = edge_pass(h4, og, od, ow, nblk)
        p = jnp.concatenate(
            [p0.reshape(-1, h_dim)[:half], p1.reshape(-1, h_dim)[:half]], 0)
        h = combine(base, p)
    return h


# speedup vs baseline: 6.8787x; 1.8127x over previous
import functools

import jax
import jax.numpy as jnp
from jax import lax
from jax.experimental import pallas as pl
from jax.experimental.pallas import tpu as pltpu
from jax.experimental.pallas import tpu_sc as plsc

NC = 2
NS = 16
NW = NC * NS
LANES = 16
HCH = 80
CH = 128
BLK = 1280
NBLK_CAP = 9
REG_CAP = NBLK_CAP * BLK
DR = 64

_MESH = dict(core_axis_name="c", subcore_axis_name="s")
_SC_PARAMS = pltpu.CompilerParams(
    use_tc_tiling_on_sc=False, needs_layout_passes=False)


def _proj_body(x_ref, wf_ref, bf_ref, o_ref):
    o_ref[...] = jnp.maximum(
        jnp.dot(x_ref[...], wf_ref[...], preferred_element_type=jnp.float32)
        + bf_ref[...], 0.0)


def _eprep_body(n_nodes, ea_ref, src_ref, dst_ref, g_ref, k_ref):
    ea = ea_ref[...]
    mx = ea[0:1, :]
    et = jnp.zeros(mx.shape, jnp.int32)
    for r in range(1, ea.shape[0]):
        row = ea[r:r + 1, :]
        gt = row > mx
        et = jnp.where(gt, r, et)
        mx = jnp.where(gt, row, mx)
    g_ref[...] = et * n_nodes + src_ref[...]
    k_ref[...] = et * n_nodes + dst_ref[...]


def _transform_body(h_ref, wr_ref, wroot_ref, bl_ref, pm_ref, h4_ref, base_ref):
    r = pl.program_id(1)
    hv = h_ref[...]

    @pl.when(r == 0)
    def _():
        base_ref[...] = jnp.dot(
            hv, wroot_ref[...], preferred_element_type=jnp.float32) + bl_ref[...]

    hr = jnp.dot(hv, wr_ref[0], preferred_element_type=jnp.float32)
    h4_ref[...] = jnp.dot(
        hr, pm_ref[...], preferred_element_type=jnp.float32).astype(jnp.bfloat16)


def _combine_body(b_ref, p_ref, o_ref):
    o_ref[...] = jnp.maximum(b_ref[...] + p_ref[...], 0.0)


def _make_hist_kernel(n4, eb):
    rows_per_tec = eb // NS
    zrows = n4 // NW

    @functools.partial(
        pl.kernel,
        out_type=jax.ShapeDtypeStruct((NW, zrows, LANES), jnp.float32),
        mesh=plsc.VectorSubcoreMesh(**_MESH),
        compiler_params=_SC_PARAMS,
        scratch_types=[
            pltpu.VMEM((rows_per_tec, HCH), jnp.int32),
            pltpu.VMEM((HCH, LANES), jnp.float32),
            pltpu.VMEM((zrows, LANES), jnp.float32),
            pltpu.VMEM_SHARED((n4, LANES), jnp.float32),
        ],
    )
    def hist_kernel(key_hbm, cnt_hbm, kidx_v, ones_v, dz_v, acc):
        c = lax.axis_index("c")
        s = lax.axis_index("s")

        def initr(i, _):
            ones_v[i, :] = jnp.ones((LANES,), jnp.float32)
            return 0
        lax.fori_loop(0, HCH, initr, 0)

        def zr(i, _):
            dz_v[i, :] = jnp.zeros((LANES,), jnp.float32)
            return 0
        lax.fori_loop(0, zrows, zr, 0)

        per_tec = n4 // NS
        for kk in range(per_tec // zrows):
            pltpu.sync_copy(dz_v, acc.at[pl.ds(s * per_tec + kk * zrows, zrows)])
        plsc.subcore_barrier()

        pltpu.sync_copy(key_hbm.at[s], kidx_v)

        def chunk(i, _):
            pltpu.sync_copy(ones_v, acc.at[kidx_v.at[i]], add=True)
            return 0
        lax.fori_loop(0, rows_per_tec, chunk, 0)
        plsc.subcore_barrier()

        dd = c * NS + s
        pltpu.sync_copy(acc.at[pl.ds(dd * zrows, zrows)], dz_v)
        pltpu.sync_copy(dz_v, cnt_hbm.at[dd])

    return hist_kernel


def _make_w_kernel(n4, eb, n_edges):
    rows_per_w = eb // NW

    @functools.partial(
        pl.kernel,
        out_type=jax.ShapeDtypeStruct((n_edges,), jnp.float32),
        mesh=plsc.VectorSubcoreMesh(**_MESH),
        compiler_params=_SC_PARAMS,
        scratch_types=[
            pltpu.VMEM((rows_per_w, HCH), jnp.int32),
            pltpu.VMEM((HCH, LANES), jnp.float32),
            pltpu.VMEM((HCH,), jnp.float32),
            pltpu.SemaphoreType.DMA,
        ],
    )
    def w_kernel(key_hbm, cnt_hbm, w_hbm, kidx_v, crows_v, wbuf_v, sem):
        c = lax.axis_index("c")
        s = lax.axis_index("s")
        wid = s * NC + c
        pltpu.sync_copy(key_hbm.at[wid], kidx_v)
        lane = lax.iota(jnp.int32, LANES)
        zeros16 = jnp.zeros((LANES,), jnp.int32)

        def chunk(i, _):
            pltpu.async_copy(cnt_hbm.at[kidx_v.at[i]], crows_v, sem).wait()

            def sub(j, _):
                ridx = j * LANES + lane
                c16 = plsc.load_gather(crows_v, [ridx, zeros16])
                wbuf_v[pl.ds(j * LANES, LANES)] = 1.0 / jnp.maximum(c16, 1.0)
                return 0
            lax.fori_loop(0, HCH // LANES, sub, 0)
            pltpu.sync_copy(
                wbuf_v, w_hbm.at[pl.ds((wid * rows_per_w + i) * HCH, HCH)])
            return 0
        lax.fori_loop(0, rows_per_w, chunk, 0)

    return w_kernel


def _make_partition_kernel(n_nodes, n_edges):
    e_per_w = n_edges // NW
    half = n_nodes // NC
    groups = e_per_w // LANES

    @functools.partial(
        pl.kernel,
        out_type=(jax.ShapeDtypeStruct((NC, NW, REG_CAP), jnp.int32),
                  jax.ShapeDtypeStruct((NC, NW, REG_CAP), jnp.int32),
                  jax.ShapeDtypeStruct((NC, NW, REG_CAP), jnp.float32),
                  jax.ShapeDtypeStruct((NC, NW, LANES), jnp.int32)),
        mesh=plsc.VectorSubcoreMesh(**_MESH),
        compiler_params=_SC_PARAMS,
        scratch_types=[
            pltpu.VMEM((e_per_w,), jnp.int32),
            pltpu.VMEM((e_per_w,), jnp.int32),
            pltpu.VMEM((e_per_w,), jnp.float32),
            pltpu.VMEM((NC, REG_CAP), jnp.int32),
            pltpu.VMEM((NC, REG_CAP), jnp.int32),
            pltpu.VMEM((NC, REG_CAP), jnp.float32),
            pltpu.VMEM((LANES,), jnp.int32),
        ],
    )
    def part_kernel(g_hbm, d_hbm, w_hbm,
                    og_hbm, od_hbm, ow_hbm, nb_hbm,
                    gb_v, db_v, wb_v, og_v, od_v, ow_v, nb_v):
        c = lax.axis_index("c")
        s = lax.axis_index("s")
        wid = s * NC + c

        pltpu.sync_copy(g_hbm.at[wid], gb_v)
        pltpu.sync_copy(d_hbm.at[wid], db_v)
        pltpu.sync_copy(w_hbm.at[wid], wb_v)

        def group(k, offs):
            o0, o1 = offs
            g16 = gb_v[pl.ds(k * LANES, LANES)]
            d16 = db_v[pl.ds(k * LANES, LANES)]
            w16 = wb_v[pl.ds(k * LANES, LANES)]
            m0 = d16 < half
            m1 = jnp.logical_not(m0)
            plsc.store_compressed(og_v.at[0, pl.ds(o0, LANES)], g16, mask=m0)
            plsc.store_compressed(od_v.at[0, pl.ds(o0, LANES)], d16, mask=m0)
            plsc.store_compressed(ow_v.at[0, pl.ds(o0, LANES)], w16, mask=m0)
            n0 = lax.reduce_max(
                plsc.all_reduce_population_count(m0), axes=(0,))
            plsc.store_compressed(og_v.at[1, pl.ds(o1, LANES)], g16, mask=m1)
            plsc.store_compressed(od_v.at[1, pl.ds(o1, LANES)], d16 - half, mask=m1)
            plsc.store_compressed(ow_v.at[1, pl.ds(o1, LANES)], w16, mask=m1)
            n1 = lax.reduce_max(
                plsc.all_reduce_population_count(m1), axes=(0,))
            return (o0 + n0, o1 + n1)

        o0, o1 = lax.fori_loop(0, groups, group, (jnp.int32(0), jnp.int32(0)))

        zi16 = jnp.zeros((LANES,), jnp.int32)
        dummyd = jnp.full((LANES,), half, jnp.int32)
        zf16 = jnp.zeros((LANES,), jnp.float32)
        for hh, off in ((0, o0), (1, o1)):
            for i in range(BLK // LANES):
                og_v[hh, pl.ds(off + i * LANES, LANES)] = zi16
                od_v[hh, pl.ds(off + i * LANES, LANES)] = dummyd
                ow_v[hh, pl.ds(off + i * LANES, LANES)] = zf16

        pltpu.sync_copy(og_v.at[0], og_hbm.at[0, wid])
        pltpu.sync_copy(od_v.at[0], od_hbm.at[0, wid])
        pltpu.sync_copy(ow_v.at[0], ow_hbm.at[0, wid])
        pltpu.sync_copy(og_v.at[1], og_hbm.at[1, wid])
        pltpu.sync_copy(od_v.at[1], od_hbm.at[1, wid])
        pltpu.sync_copy(ow_v.at[1], ow_hbm.at[1, wid])

        for hh, off in ((0, o0), (1, o1)):
            nblk = (off + (BLK - 1)) // BLK
            nb_v[:] = jnp.broadcast_to(nblk, (LANES,)).astype(jnp.int32)
            pltpu.sync_copy(nb_v, nb_hbm.at[hh, wid])

    return part_kernel


def _make_edge_kernel(n_nodes, h_dim):
    half = n_nodes // NC
    acc_rows = ((half // (NS * DR)) + 1) * NS * DR
    nsub = h_dim // LANES
    drows = acc_rows // NS // DR
    nch = BLK // CH

    @functools.partial(
        pl.kernel,
        out_type=(jax.ShapeDtypeStruct((NS * drows, DR, h_dim), jnp.float32),
                  jax.ShapeDtypeStruct((NS * drows, DR, h_dim), jnp.float32)),
        mesh=plsc.VectorSubcoreMesh(**_MESH),
        compiler_params=_SC_PARAMS,
        scratch_types=[
            pltpu.VMEM((BLK,), jnp.int32),
            pltpu.VMEM((BLK,), jnp.int32),
            pltpu.VMEM((nch, CH), jnp.int32),
            pltpu.VMEM((BLK,), jnp.float32),
            pltpu.VMEM((LANES,), jnp.int32),
            pltpu.VMEM((CH, h_dim), jnp.bfloat16),
            pltpu.VMEM((CH, h_dim), jnp.bfloat16),
            pltpu.VMEM((CH, h_dim), jnp.float32),
            pltpu.VMEM((DR, h_dim), jnp.float32),
            pltpu.SemaphoreType.DMA,
            pltpu.VMEM_SHARED((acc_rows, h_dim), jnp.float32),
        ],
    )
    def edge_kernel(h4_hbm, og_hbm, od_hbm, ow_hbm, nb_hbm, p0_hbm, p1_hbm,
                    gf_v, df_v, di_v, w_v, nb_v, rows0_v, rows1_v, sbuf_v,
                    drain_v, semg, acc):
        c = lax.axis_index("c")
        s = lax.axis_index("s")
        rows = (rows0_v, rows1_v)

        def zr(i, _):
            for t in range(nsub):
                drain_v[i, pl.ds(t * LANES, LANES)] = jnp.zeros(
                    (LANES,), jnp.float32)
            return 0
        lax.fori_loop(0, DR, zr, 0)
        for kk in range(drows):
            pltpu.sync_copy(
                drain_v, acc.at[pl.ds((s * drows + kk) * DR, DR)])
        plsc.subcore_barrier()

        for ridx in range(NW // NS):
            rg = s + NS * ridx
            pltpu.sync_copy(nb_hbm.at[c, rg], nb_v)
            nblk = lax.reduce_max(nb_v[:], axes=(0,))

            def block(bi, _):
                base = bi * BLK
                pltpu.sync_copy(og_hbm.at[c, rg, pl.ds(base, BLK)], gf_v)
                pltpu.sync_copy(od_hbm.at[c, rg, pl.ds(base, BLK)], df_v)
                pltpu.sync_copy(ow_hbm.at[c, rg, pl.ds(base, BLK)], w_v)
                for i in range(nch):
                    for j in range(CH // LANES):
                        di_v[i, pl.ds(j * LANES, LANES)] = (
                            df_v[pl.ds(i * CH + j * LANES, LANES)])

                descs_g = [None] * nch

                def gather(k):
                    return pltpu.async_copy(
                        h4_hbm.at[gf_v.at[pl.ds(k * CH, CH)]],
                        rows[k % 2], semg)

                descs_g[0] = gather(0)
                for k in range(nch):
                    descs_g[k].wait()
                    if k + 1 < nch:
                        descs_g[k + 1] = gather(k + 1)
                    rbuf = rows[k % 2]

                    def scale(j2, _, k=k, rbuf=rbuf):
                        wv = plsc.load_gather(
                            w_v, [jnp.full((LANES,), k * CH + j2, jnp.int32)])
                        for t in range(nsub // 2):
                            v32 = rbuf[j2, pl.ds(t * 2 * LANES, 2 * LANES)]
                            a, bb = plsc.unpack(
                                v32, format=plsc.PackFormat.INTERLEAVED)
                            sbuf_v[j2, pl.ds(t * 2 * LANES, LANES)] = a * wv
                            sbuf_v[j2, pl.ds(t * 2 * LANES + LANES, LANES)] = (
                                bb * wv)
                        return 0
                    lax.fori_loop(0, CH, scale, 0)
                    pltpu.sync_copy(sbuf_v, acc.at[di_v.at[k]], add=True)
                return 0
            lax.fori_loop(0, nblk, block, 0)

        plsc.subcore_barrier()
        for kk in range(drows):
            pltpu.sync_copy(acc.at[pl.ds((s * drows + kk) * DR, DR)], drain_v)

            @pl.when(c == 0)
            def _():
                pltpu.sync_copy(drain_v, p0_hbm.at[s * drows + kk])

            @pl.when(c == 1)
            def _():
                pltpu.sync_copy(drain_v, p1_hbm.at[s * drows + kk])

    return edge_kernel


def kernel(x, edge_index, edge_attr, W_f, b_f, W_rel, W_root, b):
    n, d = x.shape
    h_dim = W_f.shape[1]
    n_edges = edge_index.shape[1]
    n_rel = edge_attr.shape[1]
    n_layers = W_rel.shape[0]
    n4 = n_rel * n
    eb = n_edges // HCH
    half = n // NC
    bn = 1000
    nb = n // bn
    be = 6400
    f32 = jnp.float32

    src = edge_index[0].reshape(1, n_edges)
    dst = edge_index[1].reshape(1, n_edges)
    ea_t = edge_attr.T

    proj = pl.pallas_call(
        _proj_body,
        grid=(nb,),
        in_specs=[
            pl.BlockSpec((bn, d), lambda i: (i, 0)),
            pl.BlockSpec((d, h_dim), lambda i: (0, 0)),
            pl.BlockSpec((1, h_dim), lambda i: (0, 0)),
        ],
        out_specs=pl.BlockSpec((bn, h_dim), lambda i: (i, 0)),
        out_shape=jax.ShapeDtypeStruct((n, h_dim), f32),
    )
    h = proj(x, W_f, b_f.reshape(1, h_dim))

    eprep = pl.pallas_call(
        functools.partial(_eprep_body, n),
        grid=(n_edges // be,),
        in_specs=[
            pl.BlockSpec((n_rel, be), lambda i: (0, i)),
            pl.BlockSpec((1, be), lambda i: (0, i)),
            pl.BlockSpec((1, be), lambda i: (0, i)),
        ],
        out_specs=[
            pl.BlockSpec((1, be), lambda i: (0, i)),
            pl.BlockSpec((1, be), lambda i: (0, i)),
        ],
        out_shape=[jax.ShapeDtypeStruct((1, n_edges), jnp.int32),
                   jax.ShapeDtypeStruct((1, n_edges), jnp.int32)],
    )
    g2, k2 = eprep(ea_t, src, dst)
    k16 = k2.reshape(NS, eb // NS, HCH)
    k32 = k2.reshape(NW, eb // NW, HCH)
    g32 = g2.reshape(NW, n_edges // NW)
    d32 = dst.reshape(NW, n_edges // NW)

    cnt = _make_hist_kernel(n4, eb)(k16).reshape(n4, LANES)
    w = _make_w_kernel(n4, eb, n_edges)(k32, cnt)
    w32 = w.reshape(NW, n_edges // NW)

    og, od, ow, nblk = _make_partition_kernel(n, n_edges)(g32, d32, w32)

    import numpy as _np
    sigma = _np.empty((h_dim,), _np.int64)
    for t in range(h_dim // 32):
        for i in range(16):
            sigma[32 * t + 2 * i] = 32 * t + i
            sigma[32 * t + 2 * i + 1] = 32 * t + 16 + i
    perm_mat = jnp.asarray(_np.eye(h_dim, dtype=_np.float32)[sigma].T)

    transform = pl.pallas_call(
        _transform_body,
        grid=(nb, n_rel),
        in_specs=[
            pl.BlockSpec((bn, h_dim), lambda i, r: (i, 0)),
            pl.BlockSpec((1, h_dim, h_dim), lambda i, r: (r, 0, 0)),
            pl.BlockSpec((h_dim, h_dim), lambda i, r: (0, 0)),
            pl.BlockSpec((1, h_dim), lambda i, r: (0, 0)),
            pl.BlockSpec((h_dim, h_dim), lambda i, r: (0, 0)),
        ],
        out_specs=[
            pl.BlockSpec((bn, h_dim), lambda i, r: (r * nb + i, 0)),
            pl.BlockSpec((bn, h_dim), lambda i, r: (i, 0)),
        ],
        out_shape=[jax.ShapeDtypeStruct((n4, h_dim), jnp.bfloat16),
                   jax.ShapeDtypeStruct((n, h_dim), f32)],
    )

    combine = pl.pallas_call(
        _combine_body,
        grid=(nb,),
        in_specs=[pl.BlockSpec((bn, h_dim), lambda i: (i, 0))] * 2,
        out_specs=pl.BlockSpec((bn, h_dim), lambda i: (i, 0)),
        out_shape=jax.ShapeDtypeStruct((n, h_dim), f32),
    )

    edge_pass = _make_edge_kernel(n, h_dim)

    for l in range(n_layers):
        h4, base = transform(
            h, W_rel[l], W_root[l], b[l].reshape(1, h_dim), perm_mat)
        p0, p1 = edge_pass(h4, og, od, ow, nblk)
        p = jnp.concatenate(
            [p0.reshape(-1, h_dim)[:half], p1.reshape(-1, h_dim)[:half]], 0)
        h = combine(base, p)
    return h


# speedup vs baseline: 7.3367x; 1.0666x over previous
import functools

import jax
import jax.numpy as jnp
from jax import lax
from jax.experimental import pallas as pl
from jax.experimental.pallas import tpu as pltpu
from jax.experimental.pallas import tpu_sc as plsc

NC = 2
NS = 16
NW = NC * NS
LANES = 16
HCH = 80
CH = 128
BLK = 1280
NBLK_CAP = 9
REG_CAP = NBLK_CAP * BLK
DR = 64

_MESH = dict(core_axis_name="c", subcore_axis_name="s")
_SC_PARAMS = pltpu.CompilerParams(
    use_tc_tiling_on_sc=False, needs_layout_passes=False)


def _proj_body(x_ref, wf_ref, bf_ref, o_ref):
    o_ref[...] = jnp.maximum(
        jnp.dot(x_ref[...], wf_ref[...], preferred_element_type=jnp.float32)
        + bf_ref[...], 0.0)


def _eprep_body(n_nodes, ea_ref, src_ref, dst_ref, g_ref, k_ref):
    ea = ea_ref[...]
    mx = ea[0:1, :]
    et = jnp.zeros(mx.shape, jnp.int32)
    for r in range(1, ea.shape[0]):
        row = ea[r:r + 1, :]
        gt = row > mx
        et = jnp.where(gt, r, et)
        mx = jnp.where(gt, row, mx)
    g_ref[...] = et * n_nodes + src_ref[...]
    k_ref[...] = et * n_nodes + dst_ref[...]


def _transform_body(h_ref, wr_ref, wroot_ref, bl_ref, pm_ref, h4_ref, base_ref):
    r = pl.program_id(1)
    hv = h_ref[...]

    @pl.when(r == 0)
    def _():
        base_ref[...] = jnp.dot(
            hv, wroot_ref[...], preferred_element_type=jnp.float32) + bl_ref[...]

    hr = jnp.dot(hv, wr_ref[0], preferred_element_type=jnp.float32)
    h4_ref[...] = jnp.dot(
        hr, pm_ref[...], preferred_element_type=jnp.float32).astype(jnp.bfloat16)


def _combine_body(b_ref, p_ref, o_ref):
    o_ref[...] = jnp.maximum(b_ref[...] + p_ref[...], 0.0)


def _make_hist_kernel(n4, eb):
    rows_per_tec = eb // NS
    zrows = n4 // NW

    @functools.partial(
        pl.kernel,
        out_type=jax.ShapeDtypeStruct((NW, zrows, LANES), jnp.float32),
        mesh=plsc.VectorSubcoreMesh(**_MESH),
        compiler_params=_SC_PARAMS,
        scratch_types=[
            pltpu.VMEM((rows_per_tec, HCH), jnp.int32),
            pltpu.VMEM((HCH, LANES), jnp.float32),
            pltpu.VMEM((zrows, LANES), jnp.float32),
            pltpu.VMEM_SHARED((n4, LANES), jnp.float32),
        ],
    )
    def hist_kernel(key_hbm, cnt_hbm, kidx_v, ones_v, dz_v, acc):
        c = lax.axis_index("c")
        s = lax.axis_index("s")

        def initr(i, _):
            ones_v[i, :] = jnp.ones((LANES,), jnp.float32)
            return 0
        lax.fori_loop(0, HCH, initr, 0)

        def zr(i, _):
            dz_v[i, :] = jnp.zeros((LANES,), jnp.float32)
            return 0
        lax.fori_loop(0, zrows, zr, 0)

        per_tec = n4 // NS
        for kk in range(per_tec // zrows):
            pltpu.sync_copy(dz_v, acc.at[pl.ds(s * per_tec + kk * zrows, zrows)])
        plsc.subcore_barrier()

        pltpu.sync_copy(key_hbm.at[s], kidx_v)

        def chunk(i, _):
            pltpu.sync_copy(ones_v, acc.at[kidx_v.at[i]], add=True)
            return 0
        lax.fori_loop(0, rows_per_tec, chunk, 0)
        plsc.subcore_barrier()

        dd = c * NS + s
        pltpu.sync_copy(acc.at[pl.ds(dd * zrows, zrows)], dz_v)
        pltpu.sync_copy(dz_v, cnt_hbm.at[dd])

    return hist_kernel


def _make_w_kernel(n4, eb, n_edges):
    rows_per_w = eb // NW

    @functools.partial(
        pl.kernel,
        out_type=jax.ShapeDtypeStruct((n_edges,), jnp.float32),
        mesh=plsc.VectorSubcoreMesh(**_MESH),
        compiler_params=_SC_PARAMS,
        scratch_types=[
            pltpu.VMEM((rows_per_w, HCH), jnp.int32),
            pltpu.VMEM((HCH, LANES), jnp.float32),
            pltpu.VMEM((HCH,), jnp.float32),
            pltpu.SemaphoreType.DMA,
        ],
    )
    def w_kernel(key_hbm, cnt_hbm, w_hbm, kidx_v, crows_v, wbuf_v, sem):
        c = lax.axis_index("c")
        s = lax.axis_index("s")
        wid = s * NC + c
        pltpu.sync_copy(key_hbm.at[wid], kidx_v)
        lane = lax.iota(jnp.int32, LANES)
        zeros16 = jnp.zeros((LANES,), jnp.int32)

        def chunk(i, _):
            pltpu.async_copy(cnt_hbm.at[kidx_v.at[i]], crows_v, sem).wait()

            def sub(j, _):
                ridx = j * LANES + lane
                c16 = plsc.load_gather(crows_v, [ridx, zeros16])
                wbuf_v[pl.ds(j * LANES, LANES)] = 1.0 / jnp.maximum(c16, 1.0)
                return 0
            lax.fori_loop(0, HCH // LANES, sub, 0)
            pltpu.sync_copy(
                wbuf_v, w_hbm.at[pl.ds((wid * rows_per_w + i) * HCH, HCH)])
            return 0
        lax.fori_loop(0, rows_per_w, chunk, 0)

    return w_kernel


def _make_partition_kernel(n_nodes, n_edges):
    e_per_w = n_edges // NW
    half = n_nodes // NC
    groups = e_per_w // LANES

    @functools.partial(
        pl.kernel,
        out_type=(jax.ShapeDtypeStruct((NC, NW, REG_CAP), jnp.int32),
                  jax.ShapeDtypeStruct((NC, NW, REG_CAP), jnp.int32),
                  jax.ShapeDtypeStruct((NC, NW, REG_CAP), jnp.float32),
                  jax.ShapeDtypeStruct((NC, NW, LANES), jnp.int32)),
        mesh=plsc.VectorSubcoreMesh(**_MESH),
        compiler_params=_SC_PARAMS,
        scratch_types=[
            pltpu.VMEM((e_per_w,), jnp.int32),
            pltpu.VMEM((e_per_w,), jnp.int32),
            pltpu.VMEM((e_per_w,), jnp.float32),
            pltpu.VMEM((NC, REG_CAP), jnp.int32),
            pltpu.VMEM((NC, REG_CAP), jnp.int32),
            pltpu.VMEM((NC, REG_CAP), jnp.float32),
            pltpu.VMEM((LANES,), jnp.int32),
        ],
    )
    def part_kernel(g_hbm, d_hbm, w_hbm,
                    og_hbm, od_hbm, ow_hbm, nb_hbm,
                    gb_v, db_v, wb_v, og_v, od_v, ow_v, nb_v):
        c = lax.axis_index("c")
        s = lax.axis_index("s")
        wid = s * NC + c

        pltpu.sync_copy(g_hbm.at[wid], gb_v)
        pltpu.sync_copy(d_hbm.at[wid], db_v)
        pltpu.sync_copy(w_hbm.at[wid], wb_v)

        def group(k, offs):
            o0, o1 = offs
            g16 = gb_v[pl.ds(k * LANES, LANES)]
            d16 = db_v[pl.ds(k * LANES, LANES)]
            w16 = wb_v[pl.ds(k * LANES, LANES)]
            m0 = d16 < half
            m1 = jnp.logical_not(m0)
            plsc.store_compressed(og_v.at[0, pl.ds(o0, LANES)], g16, mask=m0)
            plsc.store_compressed(od_v.at[0, pl.ds(o0, LANES)], d16, mask=m0)
            plsc.store_compressed(ow_v.at[0, pl.ds(o0, LANES)], w16, mask=m0)
            n0 = lax.reduce_max(
                plsc.all_reduce_population_count(m0), axes=(0,))
            plsc.store_compressed(og_v.at[1, pl.ds(o1, LANES)], g16, mask=m1)
            plsc.store_compressed(od_v.at[1, pl.ds(o1, LANES)], d16 - half, mask=m1)
            plsc.store_compressed(ow_v.at[1, pl.ds(o1, LANES)], w16, mask=m1)
            n1 = lax.reduce_max(
                plsc.all_reduce_population_count(m1), axes=(0,))
            return (o0 + n0, o1 + n1)

        o0, o1 = lax.fori_loop(0, groups, group, (jnp.int32(0), jnp.int32(0)))

        zi16 = jnp.zeros((LANES,), jnp.int32)
        dummyd = jnp.full((LANES,), half, jnp.int32)
        zf16 = jnp.zeros((LANES,), jnp.float32)
        for hh, off in ((0, o0), (1, o1)):
            for i in range(BLK // LANES):
                og_v[hh, pl.ds(off + i * LANES, LANES)] = zi16
                od_v[hh, pl.ds(off + i * LANES, LANES)] = dummyd
                ow_v[hh, pl.ds(off + i * LANES, LANES)] = zf16

        pltpu.sync_copy(og_v.at[0], og_hbm.at[0, wid])
        pltpu.sync_copy(od_v.at[0], od_hbm.at[0, wid])
        pltpu.sync_copy(ow_v.at[0], ow_hbm.at[0, wid])
        pltpu.sync_copy(og_v.at[1], og_hbm.at[1, wid])
        pltpu.sync_copy(od_v.at[1], od_hbm.at[1, wid])
        pltpu.sync_copy(ow_v.at[1], ow_hbm.at[1, wid])

        for hh, off in ((0, o0), (1, o1)):
            nblk = (off + (BLK - 1)) // BLK
            nb_v[:] = jnp.broadcast_to(nblk, (LANES,)).astype(jnp.int32)
            pltpu.sync_copy(nb_v, nb_hbm.at[hh, wid])

    return part_kernel


def _make_edge_kernel(n_nodes, h_dim):
    half = n_nodes // NC
    acc_rows = ((half // (NS * DR)) + 1) * NS * DR
    nsub = h_dim // LANES
    drows = acc_rows // NS // DR
    nch = BLK // CH

    @functools.partial(
        pl.kernel,
        out_type=(jax.ShapeDtypeStruct((NS * drows, DR, h_dim), jnp.float32),
                  jax.ShapeDtypeStruct((NS * drows, DR, h_dim), jnp.float32)),
        mesh=plsc.VectorSubcoreMesh(**_MESH),
        compiler_params=_SC_PARAMS,
        scratch_types=[
            pltpu.VMEM((BLK,), jnp.int32),
            pltpu.VMEM((BLK,), jnp.int32),
            pltpu.VMEM((2 * nch, CH // 2), jnp.int32),
            pltpu.VMEM((BLK,), jnp.float32),
            pltpu.VMEM((LANES,), jnp.int32),
            pltpu.VMEM((CH, h_dim), jnp.bfloat16),
            pltpu.VMEM((CH, h_dim), jnp.bfloat16),
            pltpu.VMEM((CH // 2, h_dim), jnp.float32),
            pltpu.VMEM((CH // 2, h_dim), jnp.float32),
            pltpu.VMEM((DR, h_dim), jnp.float32),
            pltpu.SemaphoreType.DMA,
            pltpu.SemaphoreType.DMA,
            pltpu.VMEM_SHARED((acc_rows, h_dim), jnp.float32),
        ],
    )
    def edge_kernel(h4_hbm, og_hbm, od_hbm, ow_hbm, nb_hbm, p0_hbm, p1_hbm,
                    gf_v, df_v, di_v, w_v, nb_v, rows0_v, rows1_v, sbufa_v,
                    sbufb_v, drain_v, semg, sems, acc):
        c = lax.axis_index("c")
        s = lax.axis_index("s")
        rows = (rows0_v, rows1_v)

        def zr(i, _):
            for t in range(nsub):
                drain_v[i, pl.ds(t * LANES, LANES)] = jnp.zeros(
                    (LANES,), jnp.float32)
            return 0
        lax.fori_loop(0, DR, zr, 0)
        for kk in range(drows):
            pltpu.sync_copy(
                drain_v, acc.at[pl.ds((s * drows + kk) * DR, DR)])
        plsc.subcore_barrier()

        for ridx in range(NW // NS):
            rg = s + NS * ridx
            pltpu.sync_copy(nb_hbm.at[c, rg], nb_v)
            nblk = lax.reduce_max(nb_v[:], axes=(0,))

            def block(bi, _):
                base = bi * BLK
                pltpu.sync_copy(og_hbm.at[c, rg, pl.ds(base, BLK)], gf_v)
                pltpu.sync_copy(od_hbm.at[c, rg, pl.ds(base, BLK)], df_v)
                pltpu.sync_copy(ow_hbm.at[c, rg, pl.ds(base, BLK)], w_v)
                hch2 = CH // 2
                for i in range(2 * nch):
                    for j in range(hch2 // LANES):
                        di_v[i, pl.ds(j * LANES, LANES)] = (
                            df_v[pl.ds(i * hch2 + j * LANES, LANES)])

                descs_g = [None] * nch
                descs_s = [[None, None] for _ in range(nch)]

                def gather(k):
                    return pltpu.async_copy(
                        h4_hbm.at[gf_v.at[pl.ds(k * CH, CH)]],
                        rows[k % 2], semg)

                descs_g[0] = gather(0)
                for k in range(nch):
                    descs_g[k].wait()
                    if k + 1 < nch:
                        descs_g[k + 1] = gather(k + 1)
                    rbuf = rows[k % 2]
                    for hb, sb in ((0, sbufa_v), (1, sbufb_v)):
                        if k - 1 >= 0:
                            descs_s[k - 1][hb].wait()

                        def scale(j2, _, k=k, hb=hb, rbuf=rbuf, sb=sb):
                            roff = hb * hch2 + j2
                            wv = plsc.load_gather(
                                w_v,
                                [jnp.full((LANES,), k * CH + roff, jnp.int32)])
                            for t in range(nsub // 2):
                                v32 = rbuf[roff, pl.ds(t * 2 * LANES,
                                                       2 * LANES)]
                                a, bb = plsc.unpack(
                                    v32, format=plsc.PackFormat.INTERLEAVED)
                                sb[j2, pl.ds(t * 2 * LANES, LANES)] = a * wv
                                sb[j2, pl.ds(t * 2 * LANES + LANES,
                                             LANES)] = bb * wv
                            return 0
                        lax.fori_loop(0, hch2, scale, 0)
                        descs_s[k][hb] = pltpu.async_copy(
                            sb, acc.at[di_v.at[2 * k + hb]], sems, add=True)
                descs_s[nch - 1][0].wait()
                descs_s[nch - 1][1].wait()
                return 0
            lax.fori_loop(0, nblk, block, 0)

        plsc.subcore_barrier()
        for kk in range(drows):
            pltpu.sync_copy(acc.at[pl.ds((s * drows + kk) * DR, DR)], drain_v)

            @pl.when(c == 0)
            def _():
                pltpu.sync_copy(drain_v, p0_hbm.at[s * drows + kk])

            @pl.when(c == 1)
            def _():
                pltpu.sync_copy(drain_v, p1_hbm.at[s * drows + kk])

    return edge_kernel


def kernel(x, edge_index, edge_attr, W_f, b_f, W_rel, W_root, b):
    n, d = x.shape
    h_dim = W_f.shape[1]
    n_edges = edge_index.shape[1]
    n_rel = edge_attr.shape[1]
    n_layers = W_rel.shape[0]
    n4 = n_rel * n
    eb = n_edges // HCH
    half = n // NC
    bn = 1000
    nb = n // bn
    be = 6400
    f32 = jnp.float32

    src = edge_index[0].reshape(1, n_edges)
    dst = edge_index[1].reshape(1, n_edges)
    ea_t = edge_attr.T

    proj = pl.pallas_call(
        _proj_body,
        grid=(nb,),
        in_specs=[
            pl.BlockSpec((bn, d), lambda i: (i, 0)),
            pl.BlockSpec((d, h_dim), lambda i: (0, 0)),
            pl.BlockSpec((1, h_dim), lambda i: (0, 0)),
        ],
        out_specs=pl.BlockSpec((bn, h_dim), lambda i: (i, 0)),
        out_shape=jax.ShapeDtypeStruct((n, h_dim), f32),
    )
    h = proj(x, W_f, b_f.reshape(1, h_dim))

    eprep = pl.pallas_call(
        functools.partial(_eprep_body, n),
        grid=(n_edges // be,),
        in_specs=[
            pl.BlockSpec((n_rel, be), lambda i: (0, i)),
            pl.BlockSpec((1, be), lambda i: (0, i)),
            pl.BlockSpec((1, be), lambda i: (0, i)),
        ],
        out_specs=[
            pl.BlockSpec((1, be), lambda i: (0, i)),
            pl.BlockSpec((1, be), lambda i: (0, i)),
        ],
        out_shape=[jax.ShapeDtypeStruct((1, n_edges), jnp.int32),
                   jax.ShapeDtypeStruct((1, n_edges), jnp.int32)],
    )
    g2, k2 = eprep(ea_t, src, dst)
    k16 = k2.reshape(NS, eb // NS, HCH)
    k32 = k2.reshape(NW, eb // NW, HCH)
    g32 = g2.reshape(NW, n_edges // NW)
    d32 = dst.reshape(NW, n_edges // NW)

    cnt = _make_hist_kernel(n4, eb)(k16).reshape(n4, LANES)
    w = _make_w_kernel(n4, eb, n_edges)(k32, cnt)
    w32 = w.reshape(NW, n_edges // NW)

    og, od, ow, nblk = _make_partition_kernel(n, n_edges)(g32, d32, w32)

    import numpy as _np
    sigma = _np.empty((h_dim,), _np.int64)
    for t in range(h_dim // 32):
        for i in range(16):
            sigma[32 * t + 2 * i] = 32 * t + i
            sigma[32 * t + 2 * i + 1] = 32 * t + 16 + i
    perm_mat = jnp.asarray(_np.eye(h_dim, dtype=_np.float32)[sigma].T)

    transform = pl.pallas_call(
        _transform_body,
        grid=(nb, n_rel),
        in_specs=[
            pl.BlockSpec((bn, h_dim), lambda i, r: (i, 0)),
            pl.BlockSpec((1, h_dim, h_dim), lambda i, r: (r, 0, 0)),
            pl.BlockSpec((h_dim, h_dim), lambda i, r: (0, 0)),
            pl.BlockSpec((1, h_dim), lambda i, r: (0, 0)),
            pl.BlockSpec((h_dim, h_dim), lambda i, r: (0, 0)),
        ],
        out_specs=[
            pl.BlockSpec((bn, h_dim), lambda i, r: (r * nb + i, 0)),
            pl.BlockSpec((bn, h_dim), lambda i, r: (i, 0)),
        ],
        out_shape=[jax.ShapeDtypeStruct((n4, h_dim), jnp.bfloat16),
                   jax.ShapeDtypeStruct((n, h_dim), f32)],
    )

    combine = pl.pallas_call(
        _combine_body,
        grid=(nb,),
        in_specs=[pl.BlockSpec((bn, h_dim), lambda i: (i, 0))] * 2,
        out_specs=pl.BlockSpec((bn, h_dim), lambda i: (i, 0)),
        out_shape=jax.ShapeDtypeStruct((n, h_dim), f32),
    )

    edge_pass = _make_edge_kernel(n, h_dim)

    for l in range(n_layers):
        h4, base = transform(
            h, W_rel[l], W_root[l], b[l].reshape(1, h_dim), perm_mat)
        p0, p1 = edge_pass(h4, og, od, ow, nblk)
        p = jnp.concatenate(
            [p0.reshape(-1, h_dim)[:half], p1.reshape(-1, h_dim)[:half]], 0)
        h = combine(base, p)
    return h
